# baseline (speedup 1.0000x reference)
import functools

import jax
import jax.numpy as jnp
from jax import lax
from jax.experimental import pallas as pl
from jax.experimental.pallas import tpu as pltpu
from jax.experimental.pallas import tpu_sc as plsc

_N = 10000
_NP = 10240
_E = 320000
_H = 128
_DE = 16
_AD = 8

_SCI = plsc.get_sparse_core_info()
_NC = _SCI.num_cores
_NS = _SCI.num_subcores
_NW = _NC * _NS

_BN = 2000
_BE = 2560
_CH = 80

_f32 = jnp.float32


def _proj_body(h_ref, Wn_ref, A_ref, B_ref, V_ref, U_ref,
               h0_ref, td_ref, ts_ref, hu_ref):
    h0 = jnp.dot(h_ref[...], Wn_ref[...], preferred_element_type=_f32)
    h0_ref[...] = h0
    td_ref[...] = jnp.dot(h0, A_ref[...], preferred_element_type=_f32)
    ts_ref[:, :_H] = jnp.dot(h0, B_ref[...],
                             preferred_element_type=_f32).astype(_bf16)
    ts_ref[:, _H:] = jnp.dot(h0, V_ref[...],
                             preferred_element_type=_f32).astype(_bf16)
    hu_ref[...] = jnp.dot(h0, U_ref[...], preferred_element_type=_f32)


def _tc_proj(h, Wn, A0, B0, V0, U0):
    n_blk = pl.BlockSpec((_BN, _H), lambda i: (i, 0))
    w_blk = pl.BlockSpec((_H, _H), lambda i: (0, 0))
    return pl.pallas_call(
        _proj_body,
        grid=(_N // _BN,),
        in_specs=[n_blk, w_blk, w_blk, w_blk, w_blk, w_blk],
        out_specs=[n_blk, n_blk, pl.BlockSpec((_BN, 2 * _H), lambda i: (i, 0)),
                   n_blk],
        out_shape=[
            jax.ShapeDtypeStruct((_N, _H), _f32),
            jax.ShapeDtypeStruct((_N, _H), _f32),
            jax.ShapeDtypeStruct((_N, 2 * _H), _bf16),
            jax.ShapeDtypeStruct((_N, _H), _f32),
        ],
    )(h, Wn, A0, B0, V0, U0)


def _upd_proj_body(h_ref, num_ref, den_ref, hu_ref,
                   A_ref, B_ref, V_ref, U_ref,
                   h1_ref, td_ref, ts_ref, hu1_ref):
    agg = num_ref[...] / (den_ref[...] + 1e-6)
    h1 = h_ref[...] + jnp.maximum(hu_ref[...] + agg, 0.0)
    h1_ref[...] = h1
    td_ref[...] = jnp.dot(h1, A_ref[...], preferred_element_type=_f32)
    ts_ref[:, :_H] = jnp.dot(h1, B_ref[...],
                             preferred_element_type=_f32).astype(_bf16)
    ts_ref[:, _H:] = jnp.dot(h1, V_ref[...],
                             preferred_element_type=_f32).astype(_bf16)
    hu1_ref[...] = jnp.dot(h1, U_ref[...], preferred_element_type=_f32)


def _tc_upd_proj(h0, num0, den0, hu0, A1, B1, V1, U1):
    n_blk = pl.BlockSpec((_BN, _H), lambda i: (i, 0))
    w_blk = pl.BlockSpec((_H, _H), lambda i: (0, 0))
    return pl.pallas_call(
        _upd_proj_body,
        grid=(_N // _BN,),
        in_specs=[n_blk, n_blk, n_blk, n_blk, w_blk, w_blk, w_blk, w_blk],
        out_specs=[n_blk, n_blk, pl.BlockSpec((_BN, 2 * _H), lambda i: (i, 0)),
                   n_blk],
        out_shape=[
            jax.ShapeDtypeStruct((_N, _H), _f32),
            jax.ShapeDtypeStruct((_N, _H), _f32),
            jax.ShapeDtypeStruct((_N, 2 * _H), _bf16),
            jax.ShapeDtypeStruct((_N, _H), _f32),
        ],
    )(h0, num0, den0, hu0, A1, B1, V1, U1)


def _mid0_body(er_ref, gA_ref, gBV_ref, We_ref, C0_ref, C1_ref,
               msg_ref, sig_ref, eC1_ref):
    P0 = jnp.dot(We_ref[...], C0_ref[...], preferred_element_type=_f32)
    P1 = jnp.dot(We_ref[...], C1_ref[...], preferred_element_type=_f32)
    er = er_ref[...]
    ehat = (jnp.dot(er, P0, preferred_element_type=_f32)
            + gA_ref[...] + gBV_ref[:, :_H].astype(_f32))
    sig = jax.nn.sigmoid(ehat)
    sig_ref[...] = sig
    msg_ref[...] = sig * gBV_ref[:, _H:].astype(_f32)
    r = jnp.maximum(ehat, 0.0)
    eC1_ref[...] = (jnp.dot(er, P1, preferred_element_type=_f32)
                    + jnp.dot(r, C1_ref[...],
                              preferred_element_type=_f32)).astype(_bf16)


def _tc_mid0(e_raw, gA, gBV, We, C0, C1):
    e_blk = pl.BlockSpec((_BE, _H), lambda i: (i, 0))
    return pl.pallas_call(
        _mid0_body,
        grid=(_E // _BE,),
        in_specs=[
            pl.BlockSpec((_BE, _DE), lambda i: (i, 0)),
            e_blk,
            pl.BlockSpec((_BE, 2 * _H), lambda i: (i, 0)),
            pl.BlockSpec((_DE, _H), lambda i: (0, 0)),
            pl.BlockSpec((_H, _H), lambda i: (0, 0)),
            pl.BlockSpec((_H, _H), lambda i: (0, 0)),
        ],
        out_specs=[e_blk, e_blk, e_blk],
        out_shape=[
            jax.ShapeDtypeStruct((_E, _H), _f32),
            jax.ShapeDtypeStruct((_E, _H), _f32),
            jax.ShapeDtypeStruct((_E, _H), _bf16),
        ],
    )(e_raw, gA, gBV, We, C0, C1)


def _mid1_body(eC1_ref, gA_ref, gBV_ref, msg_ref, sig_ref):
    ehat = (eC1_ref[...].astype(_f32) + gA_ref[...]
            + gBV_ref[:, :_H].astype(_f32))
    sig = jax.nn.sigmoid(ehat)
    sig_ref[...] = sig
    msg_ref[...] = sig * gBV_ref[:, _H:].astype(_f32)


def _tc_mid1(eC1, gA, gBV):
    e_blk = pl.BlockSpec((_BE, _H), lambda i: (i, 0))
    return pl.pallas_call(
        _mid1_body,
        grid=(_E // _BE,),
        in_specs=[e_blk, e_blk, pl.BlockSpec((_BE, 2 * _H), lambda i: (i, 0))],
        out_specs=[e_blk, e_blk],
        out_shape=[
            jax.ShapeDtypeStruct((_E, _H), _f32),
            jax.ShapeDtypeStruct((_E, _H), _f32),
        ],
    )(eC1, gA, gBV)


def _head_body(h_ref, num_ref, den_ref, hu_ref, act_ref,
               W1h_ref, W1a_ref, b1_ref, W2_ref, b2_ref, out_ref):
    i = pl.program_id(0)
    agg = num_ref[...] / (den_ref[...] + 1e-6)
    h2 = h_ref[...] + jnp.maximum(hu_ref[...] + agg, 0.0)
    z = jnp.maximum(
        jnp.dot(h2, W1h_ref[...], preferred_element_type=_f32)
        + jnp.dot(act_ref[...], W1a_ref[...], preferred_element_type=_f32)
        + b1_ref[...], 0.0)
    y = jnp.dot(z, W2_ref[...], preferred_element_type=_f32) + b2_ref[...]

    @pl.when(i == 0)
    def _():
        out_ref[...] = jnp.zeros_like(out_ref)

    out_ref[...] += jnp.reshape(jnp.sum(y) / _N, (1, 1))


def _tc_head(h1, num1, den1, hu1, action, W1h, W1a, b1, W2, b2):
    n_blk = pl.BlockSpec((_BN, _H), lambda i: (i, 0))
    return pl.pallas_call(
        _head_body,
        grid=(_N // _BN,),
        in_specs=[
            n_blk, n_blk, n_blk, n_blk,
            pl.BlockSpec((_BN, _AD), lambda i: (i, 0)),
            pl.BlockSpec((_H, _H), lambda i: (0, 0)),
            pl.BlockSpec((_AD, _H), lambda i: (0, 0)),
            pl.BlockSpec((1, _H), lambda i: (0, 0)),
            pl.BlockSpec((_H, 1), lambda i: (0, 0)),
            pl.BlockSpec((1, 1), lambda i: (0, 0)),
        ],
        out_specs=pl.BlockSpec((1, 1), lambda i: (0, 0)),
        out_shape=jax.ShapeDtypeStruct((1, 1), _f32),
    )(h1, num1, den1, hu1, action, W1h, W1a, b1, W2, b2)


_MESH = plsc.VectorSubcoreMesh(core_axis_name="c", subcore_axis_name="s")


_GK = 5
_GCH = 80
_GSB = _GK * _GCH
_bf16 = jnp.bfloat16


def _sc_gather_body(td_hbm, ts_hbm, dst_hbm, src_hbm, gA_hbm, gBV_hbm,
                    idx_d, idx_s, bufA, bufBV, semi, semg, semw):
    c = lax.axis_index("c")
    s = lax.axis_index("s")
    wid = s * _NC + c
    ew = _E // _NW
    nsb = ew // _GSB

    def superchunk(i, carry):
        base = wid * ew + i * _GSB
        ci0 = pltpu.async_copy(dst_hbm.at[pl.ds(base, _GSB)], idx_d, semi)
        ci1 = pltpu.async_copy(src_hbm.at[pl.ds(base, _GSB)], idx_s, semi)
        ci0.wait()
        ci1.wait()
        descs = []
        for k in range(_GK):
            sl = pl.ds(k * _GCH, _GCH)
            descs.append(pltpu.async_copy(
                td_hbm.at[idx_d.at[sl]], bufA.at[sl], semg))
            descs.append(pltpu.async_copy(
                ts_hbm.at[idx_s.at[sl]], bufBV.at[sl], semg))
        for dsc in descs:
            dsc.wait()
        w0 = pltpu.async_copy(bufA, gA_hbm.at[pl.ds(base, _GSB)], semw)
        w1 = pltpu.async_copy(bufBV, gBV_hbm.at[pl.ds(base, _GSB)], semw)
        w0.wait()
        w1.wait()
        return carry

    lax.fori_loop(0, nsb, superchunk, 0)


@functools.partial(
    pl.kernel,
    out_type=[
        jax.ShapeDtypeStruct((_E, _H), _f32),
        jax.ShapeDtypeStruct((_E, _H), jnp.int32),
    ],
    mesh=_MESH,
    scratch_types=[
        pltpu.VMEM((_GSB,), jnp.int32),
        pltpu.VMEM((_GSB,), jnp.int32),
        pltpu.VMEM((_GSB, _H), _f32),
        pltpu.VMEM((_GSB, _H), jnp.int32),
        pltpu.SemaphoreType.DMA,
        pltpu.SemaphoreType.DMA,
        pltpu.SemaphoreType.DMA,
    ],
)
def _sc_gather(td_hbm, ts_hbm, dst_hbm, src_hbm, gA_hbm, gBV_hbm,
               idx_d, idx_s, bufA, bufBV, semi, semg, semw):
    _sc_gather_body(td_hbm, ts_hbm, dst_hbm, src_hbm, gA_hbm, gBV_hbm,
                    idx_d, idx_s, bufA, bufBV, semi, semg, semw)


_SK = 5
_SCH = 40
_SSB = _SK * _SCH


def _sc_scatter_stream(data_hbm, dst3_hbm, acc, dbuf, idxbuf, semd, sems, s):
    ew = _E // _NS
    nsb = ew // _SSB

    def superchunk(i, carry):
        base = s * ew + i * _SSB
        ld = pltpu.async_copy(data_hbm.at[pl.ds(base, _SSB)], dbuf, semd)
        li = pltpu.async_copy(dst3_hbm.at[pl.ds(base // _SCH, _SK)], idxbuf,
                              semd)
        ld.wait()
        li.wait()
        descs = []
        for k in range(_SK):
            descs.append(pltpu.async_copy(
                dbuf.at[pl.ds(k * _SCH, _SCH)], acc.at[idxbuf.at[k, 0]], sems,
                add=True))
        for dsc in descs:
            dsc.wait()
        return carry

    lax.fori_loop(0, nsb, superchunk, 0)


@functools.partial(
    pl.kernel,
    out_type=[
        jax.ShapeDtypeStruct((_NP, _H), _f32),
        jax.ShapeDtypeStruct((_NP, _H), _f32),
    ],
    mesh=_MESH,
    scratch_types=[
        pltpu.VMEM((_SSB, _H), _f32),
        pltpu.VMEM((_SK, 1, _SCH), jnp.int32),
        pltpu.VMEM_SHARED((_NP, _H), _f32),
        pltpu.SemaphoreType.DMA,
        pltpu.SemaphoreType.DMA,
    ],
)
def _sc_scatter(msg_hbm, sig_hbm, dst3_hbm, zeros_hbm, num_hbm, den_hbm,
                dbuf, idxbuf, acc, semd, sems):
    c = lax.axis_index("c")
    s = lax.axis_index("s")
    rows = _NP // _NS
    pltpu.sync_copy(zeros_hbm.at[pl.ds(s * rows, rows)],
                    acc.at[pl.ds(s * rows, rows)])
    plsc.subcore_barrier()

    @pl.when(c == 0)
    def _():
        _sc_scatter_stream(msg_hbm, dst3_hbm, acc, dbuf, idxbuf,
                           semd, sems, s)

    @pl.when(c == 1)
    def _():
        _sc_scatter_stream(sig_hbm, dst3_hbm, acc, dbuf, idxbuf,
                           semd, sems, s)

    plsc.subcore_barrier()

    @pl.when(c == 0)
    def _():
        pltpu.sync_copy(acc.at[pl.ds(s * rows, rows)],
                        num_hbm.at[pl.ds(s * rows, rows)])

    @pl.when(c == 1)
    def _():
        pltpu.sync_copy(acc.at[pl.ds(s * rows, rows)],
                        den_hbm.at[pl.ds(s * rows, rows)])


def kernel(h, e, edge_index, action, Wn, We, A, B, C, U, V, W1, b1, W2, b2):
    src = edge_index[0]
    dst = edge_index[1]
    dst3 = dst.reshape(_E // _SCH, 1, _SCH)
    zeros_n = jnp.zeros((_NP, _H), _f32)

    def pack(t):
        r, ccol = t.shape
        return jax.lax.bitcast_convert_type(
            t.reshape(r, ccol // 2, 2), jnp.int32)

    def unpack(t):
        r, ccol = t.shape
        return jax.lax.bitcast_convert_type(t, _bf16).reshape(r, 2 * ccol)

    h0, td0, ts0, hu0 = _tc_proj(h, Wn, A[0], B[0], V[0], U[0])
    gA0, gBV0 = _sc_gather(td0, pack(ts0), dst, src)
    msg0, sig0, eC1 = _tc_mid0(e, gA0, unpack(gBV0), We, C[0], C[1])
    num0, den0 = _sc_scatter(msg0, sig0, dst3, zeros_n)

    h1, td1, ts1, hu1 = _tc_upd_proj(h0, num0, den0, hu0,
                                     A[1], B[1], V[1], U[1])
    gA1, gBV1 = _sc_gather(td1, pack(ts1), dst, src)
    msg1, sig1 = _tc_mid1(eC1, gA1, unpack(gBV1))
    num1, den1 = _sc_scatter(msg1, sig1, dst3, zeros_n)

    return _tc_head(h1, num1, den1, hu1, action,
                    W1[:_H], W1[_H:], b1.reshape(1, _H),
                    W2, b2.reshape(1, 1))


# speedup vs baseline: 2.5712x; 2.5712x over previous
import functools

import jax
import jax.numpy as jnp
import numpy as np
from jax import lax
from jax.experimental import pallas as pl
from jax.experimental.pallas import tpu as pltpu
from jax.experimental.pallas import tpu_sc as plsc

_N = 10000
_NP = 10240
_E = 320000
_H = 128
_DE = 16
_AD = 8

_SCI = plsc.get_sparse_core_info()
_NC = _SCI.num_cores
_NS = _SCI.num_subcores
_NW = _NC * _NS

_BN = 2000
_BE = 2560
_CH = 80

_f32 = jnp.float32


_HI_MASK = np.uint32(0xFFFF0000)


def _pack_cols(x):
    u = jax.lax.bitcast_convert_type(x, jnp.uint32)
    r = (u + np.uint32(0x7FFF) + ((u >> 16) & np.uint32(1))) & _HI_MASK
    lo = r[:, : _H // 2]
    hi = r[:, _H // 2:]
    return jax.lax.bitcast_convert_type(hi | (lo >> 16), jnp.int32)


def _unpack_cols(xi):
    u = jax.lax.bitcast_convert_type(xi, jnp.uint32)
    lo = jax.lax.bitcast_convert_type(u << 16, _f32)
    hi = jax.lax.bitcast_convert_type(u & _HI_MASK, _f32)
    return jnp.concatenate([lo, hi], axis=1)

def _proj_body(h_ref, Wn_ref, A_ref, B_ref, V_ref, U_ref,
               h0_ref, td_ref, ts_ref, hu_ref):
    h0 = jnp.dot(h_ref[...], Wn_ref[...], preferred_element_type=_f32)
    h0_ref[...] = h0
    td_ref[...] = jnp.dot(h0, A_ref[...], preferred_element_type=_f32)
    ts_ref[:, : _H // 2] = _pack_cols(
        jnp.dot(h0, B_ref[...], preferred_element_type=_f32))
    ts_ref[:, _H // 2:] = _pack_cols(
        jnp.dot(h0, V_ref[...], preferred_element_type=_f32))
    hu_ref[...] = jnp.dot(h0, U_ref[...], preferred_element_type=_f32)


def _tc_proj(h, Wn, A0, B0, V0, U0):
    n_blk = pl.BlockSpec((_BN, _H), lambda i: (i, 0))
    w_blk = pl.BlockSpec((_H, _H), lambda i: (0, 0))
    return pl.pallas_call(
        _proj_body,
        grid=(_N // _BN,),
        in_specs=[n_blk, w_blk, w_blk, w_blk, w_blk, w_blk],
        out_specs=[n_blk, n_blk, pl.BlockSpec((_BN, _H), lambda i: (i, 0)),
                   n_blk],
        out_shape=[
            jax.ShapeDtypeStruct((_N, _H), _f32),
            jax.ShapeDtypeStruct((_N, _H), _f32),
            jax.ShapeDtypeStruct((_N, _H), jnp.int32),
            jax.ShapeDtypeStruct((_N, _H), _f32),
        ],
    )(h, Wn, A0, B0, V0, U0)


def _upd_proj_body(h_ref, num_ref, den_ref, hu_ref,
                   A_ref, B_ref, V_ref, U_ref,
                   h1_ref, td_ref, ts_ref, hu1_ref):
    agg = num_ref[...] / (den_ref[...] + 1e-6)
    h1 = h_ref[...] + jnp.maximum(hu_ref[...] + agg, 0.0)
    h1_ref[...] = h1
    td_ref[...] = jnp.dot(h1, A_ref[...], preferred_element_type=_f32)
    ts_ref[:, : _H // 2] = _pack_cols(
        jnp.dot(h1, B_ref[...], preferred_element_type=_f32))
    ts_ref[:, _H // 2:] = _pack_cols(
        jnp.dot(h1, V_ref[...], preferred_element_type=_f32))
    hu1_ref[...] = jnp.dot(h1, U_ref[...], preferred_element_type=_f32)


def _tc_upd_proj(h0, num0, den0, hu0, A1, B1, V1, U1):
    n_blk = pl.BlockSpec((_BN, _H), lambda i: (i, 0))
    w_blk = pl.BlockSpec((_H, _H), lambda i: (0, 0))
    return pl.pallas_call(
        _upd_proj_body,
        grid=(_N // _BN,),
        in_specs=[n_blk, n_blk, n_blk, n_blk, w_blk, w_blk, w_blk, w_blk],
        out_specs=[n_blk, n_blk, pl.BlockSpec((_BN, _H), lambda i: (i, 0)),
                   n_blk],
        out_shape=[
            jax.ShapeDtypeStruct((_N, _H), _f32),
            jax.ShapeDtypeStruct((_N, _H), _f32),
            jax.ShapeDtypeStruct((_N, _H), jnp.int32),
            jax.ShapeDtypeStruct((_N, _H), _f32),
        ],
    )(h0, num0, den0, hu0, A1, B1, V1, U1)


def _mid0_body(er_ref, gA_ref, gBV_ref, We_ref, C0_ref, C1_ref,
               msg_ref, sig_ref, eC1_ref):
    P0 = jnp.dot(We_ref[...], C0_ref[...], preferred_element_type=_f32)
    P1 = jnp.dot(We_ref[...], C1_ref[...], preferred_element_type=_f32)
    er = er_ref[...]
    ehat = (jnp.dot(er, P0, preferred_element_type=_f32)
            + gA_ref[...] + _unpack_cols(gBV_ref[:, : _H // 2]))
    sig = jax.nn.sigmoid(ehat)
    sig_ref[...] = sig
    msg_ref[...] = sig * _unpack_cols(gBV_ref[:, _H // 2:])
    r = jnp.maximum(ehat, 0.0)
    eC1_ref[...] = (jnp.dot(er, P1, preferred_element_type=_f32)
                    + jnp.dot(r, C1_ref[...],
                              preferred_element_type=_f32)).astype(_bf16)


def _tc_mid0(e_raw, gA, gBV, We, C0, C1):
    e_blk = pl.BlockSpec((_BE, _H), lambda i: (i, 0))
    return pl.pallas_call(
        _mid0_body,
        grid=(_E // _BE,),
        in_specs=[
            pl.BlockSpec((_BE, _DE), lambda i: (i, 0)),
            e_blk,
            pl.BlockSpec((_BE, _H), lambda i: (i, 0)),
            pl.BlockSpec((_DE, _H), lambda i: (0, 0)),
            pl.BlockSpec((_H, _H), lambda i: (0, 0)),
            pl.BlockSpec((_H, _H), lambda i: (0, 0)),
        ],
        out_specs=[e_blk, e_blk, e_blk],
        out_shape=[
            jax.ShapeDtypeStruct((_E, _H), _f32),
            jax.ShapeDtypeStruct((_E, _H), _f32),
            jax.ShapeDtypeStruct((_E, _H), _bf16),
        ],
    )(e_raw, gA, gBV, We, C0, C1)


def _mid1_body(eC1_ref, gA_ref, gBV_ref, msg_ref, sig_ref):
    ehat = (eC1_ref[...].astype(_f32) + gA_ref[...]
            + _unpack_cols(gBV_ref[:, : _H // 2]))
    sig = jax.nn.sigmoid(ehat)
    sig_ref[...] = sig
    msg_ref[...] = sig * _unpack_cols(gBV_ref[:, _H // 2:])


def _tc_mid1(eC1, gA, gBV):
    e_blk = pl.BlockSpec((_BE, _H), lambda i: (i, 0))
    return pl.pallas_call(
        _mid1_body,
        grid=(_E // _BE,),
        in_specs=[e_blk, e_blk, pl.BlockSpec((_BE, _H), lambda i: (i, 0))],
        out_specs=[e_blk, e_blk],
        out_shape=[
            jax.ShapeDtypeStruct((_E, _H), _f32),
            jax.ShapeDtypeStruct((_E, _H), _f32),
        ],
    )(eC1, gA, gBV)


def _head_body(h_ref, num_ref, den_ref, hu_ref, act_ref,
               W1h_ref, W1a_ref, b1_ref, W2_ref, b2_ref, out_ref):
    i = pl.program_id(0)
    agg = num_ref[...] / (den_ref[...] + 1e-6)
    h2 = h_ref[...] + jnp.maximum(hu_ref[...] + agg, 0.0)
    z = jnp.maximum(
        jnp.dot(h2, W1h_ref[...], preferred_element_type=_f32)
        + jnp.dot(act_ref[...], W1a_ref[...], preferred_element_type=_f32)
        + b1_ref[...], 0.0)
    y = jnp.dot(z, W2_ref[...], preferred_element_type=_f32) + b2_ref[...]

    @pl.when(i == 0)
    def _():
        out_ref[...] = jnp.zeros_like(out_ref)

    out_ref[...] += jnp.reshape(jnp.sum(y) / _N, (1, 1))


def _tc_head(h1, num1, den1, hu1, action, W1h, W1a, b1, W2, b2):
    n_blk = pl.BlockSpec((_BN, _H), lambda i: (i, 0))
    return pl.pallas_call(
        _head_body,
        grid=(_N // _BN,),
        in_specs=[
            n_blk, n_blk, n_blk, n_blk,
            pl.BlockSpec((_BN, _AD), lambda i: (i, 0)),
            pl.BlockSpec((_H, _H), lambda i: (0, 0)),
            pl.BlockSpec((_AD, _H), lambda i: (0, 0)),
            pl.BlockSpec((1, _H), lambda i: (0, 0)),
            pl.BlockSpec((_H, 1), lambda i: (0, 0)),
            pl.BlockSpec((1, 1), lambda i: (0, 0)),
        ],
        out_specs=pl.BlockSpec((1, 1), lambda i: (0, 0)),
        out_shape=jax.ShapeDtypeStruct((1, 1), _f32),
    )(h1, num1, den1, hu1, action, W1h, W1a, b1, W2, b2)


_MESH = plsc.VectorSubcoreMesh(core_axis_name="c", subcore_axis_name="s")


_GK = 5
_GCH = 80
_GSB = _GK * _GCH
_bf16 = jnp.bfloat16


def _sc_gather_body(td_hbm, ts_hbm, dst_hbm, src_hbm, gA_hbm, gBV_hbm,
                    idx_d, idx_s, bufA, bufBV, semi, semg, semw):
    c = lax.axis_index("c")
    s = lax.axis_index("s")
    wid = s * _NC + c
    ew = _E // _NW
    nsb = ew // _GSB

    def superchunk(i, carry):
        base = wid * ew + i * _GSB
        ci0 = pltpu.async_copy(dst_hbm.at[pl.ds(base, _GSB)], idx_d, semi)
        ci1 = pltpu.async_copy(src_hbm.at[pl.ds(base, _GSB)], idx_s, semi)
        ci0.wait()
        ci1.wait()
        descs = []
        for k in range(_GK):
            sl = pl.ds(k * _GCH, _GCH)
            descs.append(pltpu.async_copy(
                td_hbm.at[idx_d.at[sl]], bufA.at[sl], semg))
            descs.append(pltpu.async_copy(
                ts_hbm.at[idx_s.at[sl]], bufBV.at[sl], semg))
        for dsc in descs:
            dsc.wait()
        w0 = pltpu.async_copy(bufA, gA_hbm.at[pl.ds(base, _GSB)], semw)
        w1 = pltpu.async_copy(bufBV, gBV_hbm.at[pl.ds(base, _GSB)], semw)
        w0.wait()
        w1.wait()
        return carry

    lax.fori_loop(0, nsb, superchunk, 0)


@functools.partial(
    pl.kernel,
    out_type=[
        jax.ShapeDtypeStruct((_E, _H), _f32),
        jax.ShapeDtypeStruct((_E, _H), jnp.int32),
    ],
    mesh=_MESH,
    scratch_types=[
        pltpu.VMEM((_GSB,), jnp.int32),
        pltpu.VMEM((_GSB,), jnp.int32),
        pltpu.VMEM((_GSB, _H), _f32),
        pltpu.VMEM((_GSB, _H), jnp.int32),
        pltpu.SemaphoreType.DMA,
        pltpu.SemaphoreType.DMA,
        pltpu.SemaphoreType.DMA,
    ],
)
def _sc_gather(td_hbm, ts_hbm, dst_hbm, src_hbm, gA_hbm, gBV_hbm,
               idx_d, idx_s, bufA, bufBV, semi, semg, semw):
    _sc_gather_body(td_hbm, ts_hbm, dst_hbm, src_hbm, gA_hbm, gBV_hbm,
                    idx_d, idx_s, bufA, bufBV, semi, semg, semw)


_SK = 5
_SCH = 40
_SSB = _SK * _SCH


def _sc_scatter_stream(data_hbm, dst3_hbm, acc, dbuf, idxbuf, semd, sems, s):
    ew = _E // _NS
    nsb = ew // _SSB

    def superchunk(i, carry):
        base = s * ew + i * _SSB
        ld = pltpu.async_copy(data_hbm.at[pl.ds(base, _SSB)], dbuf, semd)
        li = pltpu.async_copy(dst3_hbm.at[pl.ds(base // _SCH, _SK)], idxbuf,
                              semd)
        ld.wait()
        li.wait()
        descs = []
        for k in range(_SK):
            descs.append(pltpu.async_copy(
                dbuf.at[pl.ds(k * _SCH, _SCH)], acc.at[idxbuf.at[k, 0]], sems,
                add=True))
        for dsc in descs:
            dsc.wait()
        return carry

    lax.fori_loop(0, nsb, superchunk, 0)


@functools.partial(
    pl.kernel,
    out_type=[
        jax.ShapeDtypeStruct((_NP, _H), _f32),
        jax.ShapeDtypeStruct((_NP, _H), _f32),
    ],
    mesh=_MESH,
    scratch_types=[
        pltpu.VMEM((_SSB, _H), _f32),
        pltpu.VMEM((_SK, 1, _SCH), jnp.int32),
        pltpu.VMEM_SHARED((_NP, _H), _f32),
        pltpu.SemaphoreType.DMA,
        pltpu.SemaphoreType.DMA,
    ],
)
def _sc_scatter(msg_hbm, sig_hbm, dst3_hbm, zeros_hbm, num_hbm, den_hbm,
                dbuf, idxbuf, acc, semd, sems):
    c = lax.axis_index("c")
    s = lax.axis_index("s")
    rows = _NP // _NS
    pltpu.sync_copy(zeros_hbm.at[pl.ds(s * rows, rows)],
                    acc.at[pl.ds(s * rows, rows)])
    plsc.subcore_barrier()

    @pl.when(c == 0)
    def _():
        _sc_scatter_stream(msg_hbm, dst3_hbm, acc, dbuf, idxbuf,
                           semd, sems, s)

    @pl.when(c == 1)
    def _():
        _sc_scatter_stream(sig_hbm, dst3_hbm, acc, dbuf, idxbuf,
                           semd, sems, s)

    plsc.subcore_barrier()

    @pl.when(c == 0)
    def _():
        pltpu.sync_copy(acc.at[pl.ds(s * rows, rows)],
                        num_hbm.at[pl.ds(s * rows, rows)])

    @pl.when(c == 1)
    def _():
        pltpu.sync_copy(acc.at[pl.ds(s * rows, rows)],
                        den_hbm.at[pl.ds(s * rows, rows)])


def kernel(h, e, edge_index, action, Wn, We, A, B, C, U, V, W1, b1, W2, b2):
    src = edge_index[0]
    dst = edge_index[1]
    dst3 = dst.reshape(_E // _SCH, 1, _SCH)
    zeros_n = jnp.zeros((_NP, _H), _f32)

    h0, td0, ts0, hu0 = _tc_proj(h, Wn, A[0], B[0], V[0], U[0])
    gA0, gBV0 = _sc_gather(td0, ts0, dst, src)
    msg0, sig0, eC1 = _tc_mid0(e, gA0, gBV0, We, C[0], C[1])
    num0, den0 = _sc_scatter(msg0, sig0, dst3, zeros_n)

    h1, td1, ts1, hu1 = _tc_upd_proj(h0, num0, den0, hu0,
                                     A[1], B[1], V[1], U[1])
    gA1, gBV1 = _sc_gather(td1, ts1, dst, src)
    msg1, sig1 = _tc_mid1(eC1, gA1, gBV1)
    num1, den1 = _sc_scatter(msg1, sig1, dst3, zeros_n)

    return _tc_head(h1, num1, den1, hu1, action,
                    W1[:_H], W1[_H:], b1.reshape(1, _H),
                    W2, b2.reshape(1, 1))


# speedup vs baseline: 2.8949x; 1.1259x over previous
import functools

import jax
import jax.numpy as jnp
import numpy as np
from jax import lax
from jax.experimental import pallas as pl
from jax.experimental.pallas import tpu as pltpu
from jax.experimental.pallas import tpu_sc as plsc

_N = 10000
_NP = 10240
_E = 320000
_H = 128
_DE = 16
_AD = 8

_SCI = plsc.get_sparse_core_info()
_NC = _SCI.num_cores
_NS = _SCI.num_subcores
_NW = _NC * _NS

_BN = 2000
_BE = 2560
_CH = 80

_f32 = jnp.float32


_HI_MASK = np.uint32(0xFFFF0000)


def _pack_cols(x):
    u = jax.lax.bitcast_convert_type(x, jnp.uint32)
    r = (u + np.uint32(0x7FFF) + ((u >> 16) & np.uint32(1))) & _HI_MASK
    lo = r[:, : _H // 2]
    hi = r[:, _H // 2:]
    return jax.lax.bitcast_convert_type(hi | (lo >> 16), jnp.int32)


def _unpack_cols(xi):
    u = jax.lax.bitcast_convert_type(xi, jnp.uint32)
    lo = jax.lax.bitcast_convert_type(u << 16, _f32)
    hi = jax.lax.bitcast_convert_type(u & _HI_MASK, _f32)
    return jnp.concatenate([lo, hi], axis=1)

def _proj_body(h_ref, Wn_ref, A_ref, B_ref, V_ref, U_ref,
               h0_ref, td_ref, ts_ref, hu_ref):
    h0 = jnp.dot(h_ref[...], Wn_ref[...], preferred_element_type=_f32)
    h0_ref[...] = h0
    td_ref[...] = jnp.dot(h0, A_ref[...], preferred_element_type=_f32)
    ts_ref[:, : _H // 2] = _pack_cols(
        jnp.dot(h0, B_ref[...], preferred_element_type=_f32))
    ts_ref[:, _H // 2:] = _pack_cols(
        jnp.dot(h0, V_ref[...], preferred_element_type=_f32))
    hu_ref[...] = jnp.dot(h0, U_ref[...], preferred_element_type=_f32)


def _tc_proj(h, Wn, A0, B0, V0, U0):
    n_blk = pl.BlockSpec((_BN, _H), lambda i: (i, 0))
    w_blk = pl.BlockSpec((_H, _H), lambda i: (0, 0))
    return pl.pallas_call(
        _proj_body,
        grid=(_N // _BN,),
        in_specs=[n_blk, w_blk, w_blk, w_blk, w_blk, w_blk],
        out_specs=[n_blk, n_blk, pl.BlockSpec((_BN, _H), lambda i: (i, 0)),
                   n_blk],
        out_shape=[
            jax.ShapeDtypeStruct((_N, _H), _f32),
            jax.ShapeDtypeStruct((_N, _H), _f32),
            jax.ShapeDtypeStruct((_N, _H), jnp.int32),
            jax.ShapeDtypeStruct((_N, _H), _f32),
        ],
    )(h, Wn, A0, B0, V0, U0)


def _upd_proj_body(h_ref, num_ref, den_ref, hu_ref,
                   A_ref, B_ref, V_ref, U_ref,
                   h1_ref, td_ref, ts_ref, hu1_ref):
    agg = num_ref[...] / (den_ref[...] + 1e-6)
    h1 = h_ref[...] + jnp.maximum(hu_ref[...] + agg, 0.0)
    h1_ref[...] = h1
    td_ref[...] = jnp.dot(h1, A_ref[...], preferred_element_type=_f32)
    ts_ref[:, : _H // 2] = _pack_cols(
        jnp.dot(h1, B_ref[...], preferred_element_type=_f32))
    ts_ref[:, _H // 2:] = _pack_cols(
        jnp.dot(h1, V_ref[...], preferred_element_type=_f32))
    hu1_ref[...] = jnp.dot(h1, U_ref[...], preferred_element_type=_f32)


def _tc_upd_proj(h0, num0, den0, hu0, A1, B1, V1, U1):
    n_blk = pl.BlockSpec((_BN, _H), lambda i: (i, 0))
    w_blk = pl.BlockSpec((_H, _H), lambda i: (0, 0))
    return pl.pallas_call(
        _upd_proj_body,
        grid=(_N // _BN,),
        in_specs=[n_blk, n_blk, n_blk, n_blk, w_blk, w_blk, w_blk, w_blk],
        out_specs=[n_blk, n_blk, pl.BlockSpec((_BN, _H), lambda i: (i, 0)),
                   n_blk],
        out_shape=[
            jax.ShapeDtypeStruct((_N, _H), _f32),
            jax.ShapeDtypeStruct((_N, _H), _f32),
            jax.ShapeDtypeStruct((_N, _H), jnp.int32),
            jax.ShapeDtypeStruct((_N, _H), _f32),
        ],
    )(h0, num0, den0, hu0, A1, B1, V1, U1)


def _mid0_body(er_ref, gA_ref, gBV_ref, We_ref, C0_ref, C1_ref,
               msg_ref, sig_ref, eC1_ref):
    P0 = jnp.dot(We_ref[...], C0_ref[...], preferred_element_type=_f32)
    P1 = jnp.dot(We_ref[...], C1_ref[...], preferred_element_type=_f32)
    er = er_ref[...]
    ehat = (jnp.dot(er, P0, preferred_element_type=_f32)
            + gA_ref[...] + _unpack_cols(gBV_ref[:, : _H // 2]))
    sig = jax.nn.sigmoid(ehat)
    sig_ref[...] = sig
    msg_ref[...] = sig * _unpack_cols(gBV_ref[:, _H // 2:])
    r = jnp.maximum(ehat, 0.0)
    eC1_ref[...] = (jnp.dot(er, P1, preferred_element_type=_f32)
                    + jnp.dot(r, C1_ref[...],
                              preferred_element_type=_f32)).astype(_bf16)


def _tc_mid0(e_raw, gA, gBV, We, C0, C1):
    e_blk = pl.BlockSpec((_BE, _H), lambda i: (i, 0))
    return pl.pallas_call(
        _mid0_body,
        grid=(_E // _BE,),
        in_specs=[
            pl.BlockSpec((_BE, _DE), lambda i: (i, 0)),
            e_blk,
            pl.BlockSpec((_BE, _H), lambda i: (i, 0)),
            pl.BlockSpec((_DE, _H), lambda i: (0, 0)),
            pl.BlockSpec((_H, _H), lambda i: (0, 0)),
            pl.BlockSpec((_H, _H), lambda i: (0, 0)),
        ],
        out_specs=[e_blk, e_blk, e_blk],
        out_shape=[
            jax.ShapeDtypeStruct((_E, _H), _f32),
            jax.ShapeDtypeStruct((_E, _H), _f32),
            jax.ShapeDtypeStruct((_E, _H), _bf16),
        ],
    )(e_raw, gA, gBV, We, C0, C1)


def _mid1_body(eC1_ref, gA_ref, gBV_ref, msg_ref, sig_ref):
    ehat = (eC1_ref[...].astype(_f32) + gA_ref[...]
            + _unpack_cols(gBV_ref[:, : _H // 2]))
    sig = jax.nn.sigmoid(ehat)
    sig_ref[...] = sig
    msg_ref[...] = sig * _unpack_cols(gBV_ref[:, _H // 2:])


def _tc_mid1(eC1, gA, gBV):
    e_blk = pl.BlockSpec((_BE, _H), lambda i: (i, 0))
    return pl.pallas_call(
        _mid1_body,
        grid=(_E // _BE,),
        in_specs=[e_blk, e_blk, pl.BlockSpec((_BE, _H), lambda i: (i, 0))],
        out_specs=[e_blk, e_blk],
        out_shape=[
            jax.ShapeDtypeStruct((_E, _H), _f32),
            jax.ShapeDtypeStruct((_E, _H), _f32),
        ],
    )(eC1, gA, gBV)


def _head_body(h_ref, num_ref, den_ref, hu_ref, act_ref,
               W1h_ref, W1a_ref, b1_ref, W2_ref, b2_ref, out_ref):
    i = pl.program_id(0)
    agg = num_ref[...] / (den_ref[...] + 1e-6)
    h2 = h_ref[...] + jnp.maximum(hu_ref[...] + agg, 0.0)
    z = jnp.maximum(
        jnp.dot(h2, W1h_ref[...], preferred_element_type=_f32)
        + jnp.dot(act_ref[...], W1a_ref[...], preferred_element_type=_f32)
        + b1_ref[...], 0.0)
    y = jnp.dot(z, W2_ref[...], preferred_element_type=_f32) + b2_ref[...]

    @pl.when(i == 0)
    def _():
        out_ref[...] = jnp.zeros_like(out_ref)

    out_ref[...] += jnp.reshape(jnp.sum(y) / _N, (1, 1))


def _tc_head(h1, num1, den1, hu1, action, W1h, W1a, b1, W2, b2):
    n_blk = pl.BlockSpec((_BN, _H), lambda i: (i, 0))
    return pl.pallas_call(
        _head_body,
        grid=(_N // _BN,),
        in_specs=[
            n_blk, n_blk, n_blk, n_blk,
            pl.BlockSpec((_BN, _AD), lambda i: (i, 0)),
            pl.BlockSpec((_H, _H), lambda i: (0, 0)),
            pl.BlockSpec((_AD, _H), lambda i: (0, 0)),
            pl.BlockSpec((1, _H), lambda i: (0, 0)),
            pl.BlockSpec((_H, 1), lambda i: (0, 0)),
            pl.BlockSpec((1, 1), lambda i: (0, 0)),
        ],
        out_specs=pl.BlockSpec((1, 1), lambda i: (0, 0)),
        out_shape=jax.ShapeDtypeStruct((1, 1), _f32),
    )(h1, num1, den1, hu1, action, W1h, W1a, b1, W2, b2)


_MESH = plsc.VectorSubcoreMesh(core_axis_name="c", subcore_axis_name="s")


_GK = 5
_GCH = 40
_GSB = _GK * _GCH
_bf16 = jnp.bfloat16


def _sc_gather_body(td_hbm, ts_hbm, dst_hbm, src_hbm, gA_hbm, gBV_hbm,
                    idx_d0, idx_s0, bufA0, bufBV0,
                    idx_d1, idx_s1, bufA1, bufBV1, semi, semg, semw):
    c = lax.axis_index("c")
    s = lax.axis_index("s")
    wid = s * _NC + c
    ew = _E // _NW
    nsb = ew // _GSB
    w0 = wid * ew

    def start_idx(i, idx_d, idx_s):
        pltpu.async_copy(dst_hbm.at[pl.ds(w0 + i * _GSB, _GSB)], idx_d, semi)
        pltpu.async_copy(src_hbm.at[pl.ds(w0 + i * _GSB, _GSB)], idx_s, semi)

    def wait_idx(idx_d, idx_s):
        pltpu.make_async_copy(dst_hbm.at[pl.ds(0, _GSB)], idx_d, semi).wait()
        pltpu.make_async_copy(src_hbm.at[pl.ds(0, _GSB)], idx_s, semi).wait()

    def start_gathers(idx_d, idx_s, bufA, bufBV):
        for k in range(_GK):
            sl = pl.ds(k * _GCH, _GCH)
            pltpu.async_copy(td_hbm.at[idx_d.at[sl]], bufA.at[sl], semg)
            pltpu.async_copy(ts_hbm.at[idx_s.at[sl]], bufBV.at[sl], semg)

    def wait_gathers(idx_d, idx_s, bufA, bufBV):
        for k in range(_GK):
            sl = pl.ds(k * _GCH, _GCH)
            pltpu.make_async_copy(td_hbm.at[idx_d.at[sl]], bufA.at[sl],
                                  semg).wait()
            pltpu.make_async_copy(ts_hbm.at[idx_s.at[sl]], bufBV.at[sl],
                                  semg).wait()

    def start_writes(i, bufA, bufBV):
        pltpu.async_copy(bufA, gA_hbm.at[pl.ds(w0 + i * _GSB, _GSB)], semw)
        pltpu.async_copy(bufBV, gBV_hbm.at[pl.ds(w0 + i * _GSB, _GSB)], semw)

    def wait_writes(bufA, bufBV):
        pltpu.make_async_copy(bufA, gA_hbm.at[pl.ds(0, _GSB)], semw).wait()
        pltpu.make_async_copy(bufBV, gBV_hbm.at[pl.ds(0, _GSB)], semw).wait()

    start_idx(0, idx_d0, idx_s0)

    def pair(j, carry):
        i0 = 2 * j
        wait_idx(idx_d0, idx_s0)
        start_gathers(idx_d0, idx_s0, bufA0, bufBV0)
        start_idx(i0 + 1, idx_d1, idx_s1)
        wait_gathers(idx_d0, idx_s0, bufA0, bufBV0)
        start_writes(i0, bufA0, bufBV0)
        wait_idx(idx_d1, idx_s1)
        start_gathers(idx_d1, idx_s1, bufA1, bufBV1)

        @pl.when(i0 + 2 < nsb)
        def _():
            start_idx(i0 + 2, idx_d0, idx_s0)

        wait_gathers(idx_d1, idx_s1, bufA1, bufBV1)
        wait_writes(bufA0, bufBV0)
        start_writes(i0 + 1, bufA1, bufBV1)
        wait_writes(bufA1, bufBV1)
        return carry

    lax.fori_loop(0, nsb // 2, pair, 0)


@functools.partial(
    pl.kernel,
    out_type=[
        jax.ShapeDtypeStruct((_E, _H), _f32),
        jax.ShapeDtypeStruct((_E, _H), jnp.int32),
    ],
    mesh=_MESH,
    scratch_types=[
        pltpu.VMEM((_GSB,), jnp.int32),
        pltpu.VMEM((_GSB,), jnp.int32),
        pltpu.VMEM((_GSB, _H), _f32),
        pltpu.VMEM((_GSB, _H), jnp.int32),
        pltpu.VMEM((_GSB,), jnp.int32),
        pltpu.VMEM((_GSB,), jnp.int32),
        pltpu.VMEM((_GSB, _H), _f32),
        pltpu.VMEM((_GSB, _H), jnp.int32),
        pltpu.SemaphoreType.DMA,
        pltpu.SemaphoreType.DMA,
        pltpu.SemaphoreType.DMA,
    ],
)
def _sc_gather(td_hbm, ts_hbm, dst_hbm, src_hbm, gA_hbm, gBV_hbm,
               idx_d0, idx_s0, bufA0, bufBV0,
               idx_d1, idx_s1, bufA1, bufBV1, semi, semg, semw):
    _sc_gather_body(td_hbm, ts_hbm, dst_hbm, src_hbm, gA_hbm, gBV_hbm,
                    idx_d0, idx_s0, bufA0, bufBV0,
                    idx_d1, idx_s1, bufA1, bufBV1, semi, semg, semw)


_SK = 2
_SCH = 80
_SSB = _SK * _SCH


def _sc_scatter_stream(data_hbm, dst3_hbm, acc, dbuf0, idx0, dbuf1, idx1,
                       semd, sems, s):
    ew = _E // _NS
    nsb = ew // _SSB
    base0 = s * ew

    def start_load(i, dbuf, idxbuf):
        pltpu.async_copy(data_hbm.at[pl.ds(base0 + i * _SSB, _SSB)], dbuf,
                         semd)
        pltpu.async_copy(
            dst3_hbm.at[pl.ds((base0 + i * _SSB) // _SCH, _SK)], idxbuf, semd)

    def wait_load(dbuf, idxbuf):
        pltpu.make_async_copy(data_hbm.at[pl.ds(0, _SSB)], dbuf, semd).wait()
        pltpu.make_async_copy(dst3_hbm.at[pl.ds(0, _SK)], idxbuf, semd).wait()

    def scat(dbuf, idxbuf):
        for k in range(_SK):
            pltpu.async_copy(dbuf.at[pl.ds(k * _SCH, _SCH)],
                             acc.at[idxbuf.at[k, 0]], sems, add=True)
        for k in range(_SK):
            pltpu.make_async_copy(dbuf.at[pl.ds(k * _SCH, _SCH)],
                                  acc.at[idxbuf.at[k, 0]], sems).wait()

    start_load(0, dbuf0, idx0)

    def pair(j, carry):
        i0 = 2 * j
        wait_load(dbuf0, idx0)
        start_load(i0 + 1, dbuf1, idx1)
        scat(dbuf0, idx0)
        start_load(i0 + 2, dbuf0, idx0)
        wait_load(dbuf1, idx1)
        scat(dbuf1, idx1)
        return carry

    lax.fori_loop(0, (nsb - 1) // 2, pair, 0)
    wait_load(dbuf0, idx0)
    scat(dbuf0, idx0)


@functools.partial(
    pl.kernel,
    out_type=[
        jax.ShapeDtypeStruct((_NP, _H), _f32),
        jax.ShapeDtypeStruct((_NP, _H), _f32),
    ],
    mesh=_MESH,
    scratch_types=[
        pltpu.VMEM((_SSB, _H), _f32),
        pltpu.VMEM((_SK, 1, _SCH), jnp.int32),
        pltpu.VMEM((_SSB, _H), _f32),
        pltpu.VMEM((_SK, 1, _SCH), jnp.int32),
        pltpu.VMEM_SHARED((_NP, _H), _f32),
        pltpu.SemaphoreType.DMA,
        pltpu.SemaphoreType.DMA,
    ],
)
def _sc_scatter(msg_hbm, sig_hbm, dst3_hbm, zeros_hbm, num_hbm, den_hbm,
                dbuf0, idx0, dbuf1, idx1, acc, semd, sems):
    c = lax.axis_index("c")
    s = lax.axis_index("s")
    rows = _NP // _NS
    pltpu.sync_copy(zeros_hbm.at[pl.ds(s * rows, rows)],
                    acc.at[pl.ds(s * rows, rows)])
    plsc.subcore_barrier()

    @pl.when(c == 0)
    def _():
        _sc_scatter_stream(msg_hbm, dst3_hbm, acc, dbuf0, idx0, dbuf1, idx1,
                           semd, sems, s)

    @pl.when(c == 1)
    def _():
        _sc_scatter_stream(sig_hbm, dst3_hbm, acc, dbuf0, idx0, dbuf1, idx1,
                           semd, sems, s)

    plsc.subcore_barrier()

    @pl.when(c == 0)
    def _():
        pltpu.sync_copy(acc.at[pl.ds(s * rows, rows)],
                        num_hbm.at[pl.ds(s * rows, rows)])

    @pl.when(c == 1)
    def _():
        pltpu.sync_copy(acc.at[pl.ds(s * rows, rows)],
                        den_hbm.at[pl.ds(s * rows, rows)])


def kernel(h, e, edge_index, action, Wn, We, A, B, C, U, V, W1, b1, W2, b2):
    src = edge_index[0]
    dst = edge_index[1]
    dst3 = dst.reshape(_E // _SCH, 1, _SCH)
    zeros_n = jnp.zeros((_NP, _H), _f32)

    h0, td0, ts0, hu0 = _tc_proj(h, Wn, A[0], B[0], V[0], U[0])
    gA0, gBV0 = _sc_gather(td0, ts0, dst, src)
    msg0, sig0, eC1 = _tc_mid0(e, gA0, gBV0, We, C[0], C[1])
    num0, den0 = _sc_scatter(msg0, sig0, dst3, zeros_n)

    h1, td1, ts1, hu1 = _tc_upd_proj(h0, num0, den0, hu0,
                                     A[1], B[1], V[1], U[1])
    gA1, gBV1 = _sc_gather(td1, ts1, dst, src)
    msg1, sig1 = _tc_mid1(eC1, gA1, gBV1)
    num1, den1 = _sc_scatter(msg1, sig1, dst3, zeros_n)

    return _tc_head(h1, num1, den1, hu1, action,
                    W1[:_H], W1[_H:], b1.reshape(1, _H),
                    W2, b2.reshape(1, 1))


# speedup vs baseline: 2.9896x; 1.0327x over previous
import functools

import jax
import jax.numpy as jnp
import numpy as np
from jax import lax
from jax.experimental import pallas as pl
from jax.experimental.pallas import tpu as pltpu
from jax.experimental.pallas import tpu_sc as plsc

_N = 10000
_NP = 10240
_E = 320000
_EH = _E // 2
_H = 128
_DE = 16
_AD = 8

_SCI = plsc.get_sparse_core_info()
_NC = _SCI.num_cores
_NS = _SCI.num_subcores
_NW = _NC * _NS

_BN = 2000
_BE = 2000

_f32 = jnp.float32
_bf16 = jnp.bfloat16


_HI_MASK = np.uint32(0xFFFF0000)


def _pack_cols(x):
    u = jax.lax.bitcast_convert_type(x, jnp.uint32)
    r = (u + np.uint32(0x7FFF) + ((u >> 16) & np.uint32(1))) & _HI_MASK
    lo = r[:, : _H // 2]
    hi = r[:, _H // 2:]
    return jax.lax.bitcast_convert_type(hi | (lo >> 16), jnp.int32)


def _unpack_cols(xi):
    u = jax.lax.bitcast_convert_type(xi, jnp.uint32)
    lo = jax.lax.bitcast_convert_type(u << 16, _f32)
    hi = jax.lax.bitcast_convert_type(u & _HI_MASK, _f32)
    return jnp.concatenate([lo, hi], axis=1)


def _proj_tables(h0, A_ref, B_ref, V_ref, U_ref, td_ref, ts_ref, hu_ref):
    td_ref[...] = jnp.dot(h0, A_ref[...], preferred_element_type=_f32)
    ts_ref[:, : _H // 2] = _pack_cols(
        jnp.dot(h0, B_ref[...], preferred_element_type=_f32))
    ts_ref[:, _H // 2:] = _pack_cols(
        jnp.dot(h0, V_ref[...], preferred_element_type=_f32))
    hu_ref[...] = jnp.dot(h0, U_ref[...], preferred_element_type=_f32)


def _proj_body(h_ref, Wn_ref, A_ref, B_ref, V_ref, U_ref,
               h0_ref, td_ref, ts_ref, hu_ref):
    h0 = jnp.dot(h_ref[...], Wn_ref[...], preferred_element_type=_f32)
    h0_ref[...] = h0
    _proj_tables(h0, A_ref, B_ref, V_ref, U_ref, td_ref, ts_ref, hu_ref)


def _tc_proj(h, Wn, A0, B0, V0, U0):
    n_blk = pl.BlockSpec((_BN, _H), lambda i: (i, 0))
    w_blk = pl.BlockSpec((_H, _H), lambda i: (0, 0))
    return pl.pallas_call(
        _proj_body,
        grid=(_N // _BN,),
        in_specs=[n_blk, w_blk, w_blk, w_blk, w_blk, w_blk],
        out_specs=[n_blk, n_blk, pl.BlockSpec((_BN, _H), lambda i: (i, 0)),
                   n_blk],
        out_shape=[
            jax.ShapeDtypeStruct((_N, _H), _f32),
            jax.ShapeDtypeStruct((_N, _H), _f32),
            jax.ShapeDtypeStruct((_N, _H), jnp.int32),
            jax.ShapeDtypeStruct((_N, _H), _f32),
        ],
    )(h, Wn, A0, B0, V0, U0)


def _upd_proj_body(h_ref, na_ref, nb_ref, da_ref, db_ref, hu_ref,
                   A_ref, B_ref, V_ref, U_ref,
                   h1_ref, td_ref, ts_ref, hu1_ref):
    num = na_ref[...] + nb_ref[...]
    den = da_ref[...] + db_ref[...]
    h1 = h_ref[...] + jnp.maximum(hu_ref[...] + num / (den + 1e-6), 0.0)
    h1_ref[...] = h1
    _proj_tables(h1, A_ref, B_ref, V_ref, U_ref, td_ref, ts_ref, hu1_ref)


def _tc_upd_proj(h0, numa, numb, dena, denb, hu0, A1, B1, V1, U1):
    n_blk = pl.BlockSpec((_BN, _H), lambda i: (i, 0))
    w_blk = pl.BlockSpec((_H, _H), lambda i: (0, 0))
    return pl.pallas_call(
        _upd_proj_body,
        grid=(_N // _BN,),
        in_specs=[n_blk, n_blk, n_blk, n_blk, n_blk, n_blk,
                  w_blk, w_blk, w_blk, w_blk],
        out_specs=[n_blk, n_blk, pl.BlockSpec((_BN, _H), lambda i: (i, 0)),
                   n_blk],
        out_shape=[
            jax.ShapeDtypeStruct((_N, _H), _f32),
            jax.ShapeDtypeStruct((_N, _H), _f32),
            jax.ShapeDtypeStruct((_N, _H), jnp.int32),
            jax.ShapeDtypeStruct((_N, _H), _f32),
        ],
    )(h0, numa, numb, dena, denb, hu0, A1, B1, V1, U1)


def _mid0_body(er_ref, gA_ref, gBV_ref, We_ref, C0_ref, C1_ref,
               msg_ref, sig_ref, eC1_ref):
    P0 = jnp.dot(We_ref[...], C0_ref[...], preferred_element_type=_f32)
    P1 = jnp.dot(We_ref[...], C1_ref[...], preferred_element_type=_f32)
    er = er_ref[...]
    ehat = (jnp.dot(er, P0, preferred_element_type=_f32)
            + gA_ref[...] + _unpack_cols(gBV_ref[:, : _H // 2]))
    sig = jax.nn.sigmoid(ehat)
    sig_ref[...] = sig
    msg_ref[...] = sig * _unpack_cols(gBV_ref[:, _H // 2:])
    r = jnp.maximum(ehat, 0.0)
    eC1_ref[...] = (jnp.dot(er, P1, preferred_element_type=_f32)
                    + jnp.dot(r, C1_ref[...],
                              preferred_element_type=_f32)).astype(_bf16)


def _make_mid0(half):
    off = half * (_EH // _BE)
    e_blk = pl.BlockSpec((_BE, _H), lambda i: (i, 0))

    def call(e_raw, gA, gBV, We, C0, C1):
        return pl.pallas_call(
            _mid0_body,
            grid=(_EH // _BE,),
            in_specs=[
                pl.BlockSpec((_BE, _DE), lambda i: (i + off, 0)),
                e_blk,
                pl.BlockSpec((_BE, _H), lambda i: (i, 0)),
                pl.BlockSpec((_DE, _H), lambda i: (0, 0)),
                pl.BlockSpec((_H, _H), lambda i: (0, 0)),
                pl.BlockSpec((_H, _H), lambda i: (0, 0)),
            ],
            out_specs=[e_blk, e_blk, e_blk],
            out_shape=[
                jax.ShapeDtypeStruct((_EH, _H), _f32),
                jax.ShapeDtypeStruct((_EH, _H), _f32),
                jax.ShapeDtypeStruct((_EH, _H), _bf16),
            ],
        )(e_raw, gA, gBV, We, C0, C1)

    return call


_tc_mid0_h = (_make_mid0(0), _make_mid0(1))


def _mid1_body(eC1_ref, gA_ref, gBV_ref, msg_ref, sig_ref):
    ehat = (eC1_ref[...].astype(_f32) + gA_ref[...]
            + _unpack_cols(gBV_ref[:, : _H // 2]))
    sig = jax.nn.sigmoid(ehat)
    sig_ref[...] = sig
    msg_ref[...] = sig * _unpack_cols(gBV_ref[:, _H // 2:])


def _tc_mid1(eC1_h, gA, gBV):
    e_blk = pl.BlockSpec((_BE, _H), lambda i: (i, 0))
    return pl.pallas_call(
        _mid1_body,
        grid=(_EH // _BE,),
        in_specs=[e_blk, e_blk, pl.BlockSpec((_BE, _H), lambda i: (i, 0))],
        out_specs=[e_blk, e_blk],
        out_shape=[
            jax.ShapeDtypeStruct((_EH, _H), _f32),
            jax.ShapeDtypeStruct((_EH, _H), _f32),
        ],
    )(eC1_h, gA, gBV)


def _head_body(h_ref, na_ref, nb_ref, da_ref, db_ref, hu_ref, act_ref,
               W1h_ref, W1a_ref, b1_ref, W2_ref, b2_ref, out_ref):
    i = pl.program_id(0)
    num = na_ref[...] + nb_ref[...]
    den = da_ref[...] + db_ref[...]
    h2 = h_ref[...] + jnp.maximum(hu_ref[...] + num / (den + 1e-6), 0.0)
    z = jnp.maximum(
        jnp.dot(h2, W1h_ref[...], preferred_element_type=_f32)
        + jnp.dot(act_ref[...], W1a_ref[...], preferred_element_type=_f32)
        + b1_ref[...], 0.0)
    y = jnp.dot(z, W2_ref[...], preferred_element_type=_f32) + b2_ref[...]

    @pl.when(i == 0)
    def _():
        out_ref[...] = jnp.zeros_like(out_ref)

    out_ref[...] += jnp.reshape(jnp.sum(y) / _N, (1, 1))


def _tc_head(h1, numa, numb, dena, denb, hu1, action, W1h, W1a, b1, W2, b2):
    n_blk = pl.BlockSpec((_BN, _H), lambda i: (i, 0))
    return pl.pallas_call(
        _head_body,
        grid=(_N // _BN,),
        in_specs=[
            n_blk, n_blk, n_blk, n_blk, n_blk, n_blk,
            pl.BlockSpec((_BN, _AD), lambda i: (i, 0)),
            pl.BlockSpec((_H, _H), lambda i: (0, 0)),
            pl.BlockSpec((_AD, _H), lambda i: (0, 0)),
            pl.BlockSpec((1, _H), lambda i: (0, 0)),
            pl.BlockSpec((_H, 1), lambda i: (0, 0)),
            pl.BlockSpec((1, 1), lambda i: (0, 0)),
        ],
        out_specs=pl.BlockSpec((1, 1), lambda i: (0, 0)),
        out_shape=jax.ShapeDtypeStruct((1, 1), _f32),
    )(h1, numa, numb, dena, denb, hu1, action, W1h, W1a, b1, W2, b2)


_MESH = plsc.VectorSubcoreMesh(core_axis_name="c", subcore_axis_name="s")

_GK = 5
_GCH = 40
_GSB = _GK * _GCH


def _make_gather(eoff, esz):
    ew = esz // _NW
    nsb = ew // _GSB

    def body(td_hbm, ts_hbm, dst_hbm, src_hbm, gA_hbm, gBV_hbm,
             idx_d0, idx_s0, bufA0, bufBV0,
             idx_d1, idx_s1, bufA1, bufBV1, semi, semg, semw):
        c = lax.axis_index("c")
        s = lax.axis_index("s")
        wid = s * _NC + c
        w0in = eoff + wid * ew
        w0out = wid * ew

        def start_idx(i, idx_d, idx_s):
            pltpu.async_copy(dst_hbm.at[pl.ds(w0in + i * _GSB, _GSB)], idx_d,
                             semi)
            pltpu.async_copy(src_hbm.at[pl.ds(w0in + i * _GSB, _GSB)], idx_s,
                             semi)

        def wait_idx(idx_d, idx_s):
            pltpu.make_async_copy(dst_hbm.at[pl.ds(0, _GSB)], idx_d,
                                  semi).wait()
            pltpu.make_async_copy(src_hbm.at[pl.ds(0, _GSB)], idx_s,
                                  semi).wait()

        def start_gathers(idx_d, idx_s, bufA, bufBV):
            for k in range(_GK):
                sl = pl.ds(k * _GCH, _GCH)
                pltpu.async_copy(td_hbm.at[idx_d.at[sl]], bufA.at[sl], semg)
                pltpu.async_copy(ts_hbm.at[idx_s.at[sl]], bufBV.at[sl], semg)

        def wait_gathers(idx_d, idx_s, bufA, bufBV):
            for k in range(_GK):
                sl = pl.ds(k * _GCH, _GCH)
                pltpu.make_async_copy(td_hbm.at[idx_d.at[sl]], bufA.at[sl],
                                      semg).wait()
                pltpu.make_async_copy(ts_hbm.at[idx_s.at[sl]], bufBV.at[sl],
                                      semg).wait()

        def start_writes(i, bufA, bufBV):
            pltpu.async_copy(bufA, gA_hbm.at[pl.ds(w0out + i * _GSB, _GSB)],
                             semw)
            pltpu.async_copy(bufBV, gBV_hbm.at[pl.ds(w0out + i * _GSB, _GSB)],
                             semw)

        def wait_writes(bufA, bufBV):
            pltpu.make_async_copy(bufA, gA_hbm.at[pl.ds(0, _GSB)],
                                  semw).wait()
            pltpu.make_async_copy(bufBV, gBV_hbm.at[pl.ds(0, _GSB)],
                                  semw).wait()

        start_idx(0, idx_d0, idx_s0)

        def pair(j, carry):
            i0 = 2 * j
            wait_idx(idx_d0, idx_s0)
            start_gathers(idx_d0, idx_s0, bufA0, bufBV0)
            start_idx(i0 + 1, idx_d1, idx_s1)
            wait_gathers(idx_d0, idx_s0, bufA0, bufBV0)
            start_writes(i0, bufA0, bufBV0)
            wait_idx(idx_d1, idx_s1)
            start_gathers(idx_d1, idx_s1, bufA1, bufBV1)

            @pl.when(i0 + 2 < nsb)
            def _():
                start_idx(i0 + 2, idx_d0, idx_s0)

            wait_gathers(idx_d1, idx_s1, bufA1, bufBV1)
            wait_writes(bufA0, bufBV0)
            start_writes(i0 + 1, bufA1, bufBV1)
            wait_writes(bufA1, bufBV1)
            return carry

        lax.fori_loop(0, nsb // 2, pair, 0)
        if nsb % 2:
            wait_idx(idx_d0, idx_s0)
            start_gathers(idx_d0, idx_s0, bufA0, bufBV0)
            wait_gathers(idx_d0, idx_s0, bufA0, bufBV0)
            start_writes(nsb - 1, bufA0, bufBV0)
            wait_writes(bufA0, bufBV0)

    return pl.kernel(
        body,
        out_type=[
            jax.ShapeDtypeStruct((esz, _H), _f32),
            jax.ShapeDtypeStruct((esz, _H), jnp.int32),
        ],
        mesh=_MESH,
        scratch_types=[
            pltpu.VMEM((_GSB,), jnp.int32),
            pltpu.VMEM((_GSB,), jnp.int32),
            pltpu.VMEM((_GSB, _H), _f32),
            pltpu.VMEM((_GSB, _H), jnp.int32),
            pltpu.VMEM((_GSB,), jnp.int32),
            pltpu.VMEM((_GSB,), jnp.int32),
            pltpu.VMEM((_GSB, _H), _f32),
            pltpu.VMEM((_GSB, _H), jnp.int32),
            pltpu.SemaphoreType.DMA,
            pltpu.SemaphoreType.DMA,
            pltpu.SemaphoreType.DMA,
        ],
    )


_sc_gather_h = (_make_gather(0, _EH), _make_gather(_EH, _EH))

_SK = 2
_SCH = 40
_SSB = _SK * _SCH


def _make_scatter(eoff, esz):
    ew = esz // _NS
    nsb = ew // _SSB
    rows = _NP // _NS

    def stream(data_hbm, dst3_hbm, acc, dbuf0, idx0, dbuf1, idx1,
               semd, sems, s):
        base0 = s * ew

        def start_load(i, dbuf, idxbuf):
            pltpu.async_copy(data_hbm.at[pl.ds(base0 + i * _SSB, _SSB)], dbuf,
                             semd)
            pltpu.async_copy(
                dst3_hbm.at[pl.ds((eoff + base0 + i * _SSB) // _SCH, _SK)],
                idxbuf, semd)

        def wait_load(dbuf, idxbuf):
            pltpu.make_async_copy(data_hbm.at[pl.ds(0, _SSB)], dbuf,
                                  semd).wait()
            pltpu.make_async_copy(dst3_hbm.at[pl.ds(0, _SK)], idxbuf,
                                  semd).wait()

        def scat(dbuf, idxbuf):
            for k in range(_SK):
                pltpu.async_copy(dbuf.at[pl.ds(k * _SCH, _SCH)],
                                 acc.at[idxbuf.at[k, 0]], sems, add=True)
            for k in range(_SK):
                pltpu.make_async_copy(dbuf.at[pl.ds(k * _SCH, _SCH)],
                                      acc.at[idxbuf.at[k, 0]], sems).wait()

        start_load(0, dbuf0, idx0)

        def pairfn(j, carry):
            i0 = 2 * j
            wait_load(dbuf0, idx0)
            start_load(i0 + 1, dbuf1, idx1)
            scat(dbuf0, idx0)

            @pl.when(i0 + 2 < nsb)
            def _():
                start_load(i0 + 2, dbuf0, idx0)

            wait_load(dbuf1, idx1)
            scat(dbuf1, idx1)
            return carry

        lax.fori_loop(0, nsb // 2, pairfn, 0)
        if nsb % 2:
            wait_load(dbuf0, idx0)
            scat(dbuf0, idx0)

    def body(msg_hbm, sig_hbm, dst3_hbm, zeros_hbm, num_hbm, den_hbm,
             dbuf0, idx0, dbuf1, idx1, acc, semd, sems):
        c = lax.axis_index("c")
        s = lax.axis_index("s")
        pltpu.sync_copy(zeros_hbm.at[pl.ds(s * rows, rows)],
                        acc.at[pl.ds(s * rows, rows)])
        plsc.subcore_barrier()

        @pl.when(c == 0)
        def _():
            stream(msg_hbm, dst3_hbm, acc, dbuf0, idx0, dbuf1, idx1,
                   semd, sems, s)

        @pl.when(c == 1)
        def _():
            stream(sig_hbm, dst3_hbm, acc, dbuf0, idx0, dbuf1, idx1,
                   semd, sems, s)

        plsc.subcore_barrier()

        @pl.when(c == 0)
        def _():
            pltpu.sync_copy(acc.at[pl.ds(s * rows, rows)],
                            num_hbm.at[pl.ds(s * rows, rows)])

        @pl.when(c == 1)
        def _():
            pltpu.sync_copy(acc.at[pl.ds(s * rows, rows)],
                            den_hbm.at[pl.ds(s * rows, rows)])

    return pl.kernel(
        body,
        out_type=[
            jax.ShapeDtypeStruct((_NP, _H), _f32),
            jax.ShapeDtypeStruct((_NP, _H), _f32),
        ],
        mesh=_MESH,
        scratch_types=[
            pltpu.VMEM((_SSB, _H), _f32),
            pltpu.VMEM((_SK, 1, _SCH), jnp.int32),
            pltpu.VMEM((_SSB, _H), _f32),
            pltpu.VMEM((_SK, 1, _SCH), jnp.int32),
            pltpu.VMEM_SHARED((_NP, _H), _f32),
            pltpu.SemaphoreType.DMA,
            pltpu.SemaphoreType.DMA,
        ],
    )


_sc_scatter_h = (_make_scatter(0, _EH), _make_scatter(_EH, _EH))


def _layer_edges(e_raw, eC1_h, td, ts, dst, src, dst3, zeros_n, We, C0, C1,
                 layer):
    gA = [None, None]
    gBV = [None, None]
    out = [None, None]
    nd = [None, None]
    gA[0], gBV[0] = _sc_gather_h[0](td, ts, dst, src)
    for hf in (0, 1):
        if hf == 0:
            gA[1], gBV[1] = _sc_gather_h[1](td, ts, dst, src)
        if layer == 0:
            out[hf] = _tc_mid0_h[hf](e_raw, gA[hf], gBV[hf], We, C0, C1)
        else:
            out[hf] = _tc_mid1(eC1_h[hf], gA[hf], gBV[hf])
        nd[hf] = _sc_scatter_h[hf](out[hf][0], out[hf][1], dst3, zeros_n)
    if layer == 0:
        return (nd[0][0], nd[1][0], nd[0][1], nd[1][1],
                (out[0][2], out[1][2]))
    return nd[0][0], nd[1][0], nd[0][1], nd[1][1]


def kernel(h, e, edge_index, action, Wn, We, A, B, C, U, V, W1, b1, W2, b2):
    src = edge_index[0]
    dst = edge_index[1]
    dst3 = dst.reshape(_E // _SCH, 1, _SCH)
    zeros_n = jnp.zeros((_NP, _H), _f32)

    h0, td0, ts0, hu0 = _tc_proj(h, Wn, A[0], B[0], V[0], U[0])
    na0, nb0, da0, db0, eC1_h = _layer_edges(
        e, None, td0, ts0, dst, src, dst3, zeros_n, We, C[0], C[1], 0)

    h1, td1, ts1, hu1 = _tc_upd_proj(h0, na0, nb0, da0, db0, hu0,
                                     A[1], B[1], V[1], U[1])
    na1, nb1, da1, db1 = _layer_edges(
        None, eC1_h, td1, ts1, dst, src, dst3, zeros_n, We, C[0], C[1], 1)

    return _tc_head(h1, na1, nb1, da1, db1, hu1, action,
                    W1[:_H], W1[_H:], b1.reshape(1, _H),
                    W2, b2.reshape(1, 1))


# speedup vs baseline: 3.0029x; 1.0045x over previous
import functools

import jax
import jax.numpy as jnp
import numpy as np
from jax import lax
from jax.experimental import pallas as pl
from jax.experimental.pallas import tpu as pltpu
from jax.experimental.pallas import tpu_sc as plsc

_N = 10000
_NP = 10240
_E = 320000
_EH = _E // 2
_H = 128
_DE = 16
_AD = 8

_SCI = plsc.get_sparse_core_info()
_NC = _SCI.num_cores
_NS = _SCI.num_subcores
_NW = _NC * _NS

_BN = 2000
_BE = 2000

_f32 = jnp.float32
_bf16 = jnp.bfloat16


_HI_MASK = np.uint32(0xFFFF0000)


def _pack_cols(x):
    u = jax.lax.bitcast_convert_type(x, jnp.uint32)
    r = (u + np.uint32(0x7FFF) + ((u >> 16) & np.uint32(1))) & _HI_MASK
    lo = r[:, : _H // 2]
    hi = r[:, _H // 2:]
    return jax.lax.bitcast_convert_type(hi | (lo >> 16), jnp.int32)


def _unpack_cols(xi):
    u = jax.lax.bitcast_convert_type(xi, jnp.uint32)
    lo = jax.lax.bitcast_convert_type(u << 16, _f32)
    hi = jax.lax.bitcast_convert_type(u & _HI_MASK, _f32)
    return jnp.concatenate([lo, hi], axis=1)


def _proj_tables(h0, A_ref, B_ref, V_ref, U_ref, td_ref, ts_ref, hu_ref):
    td_ref[...] = jnp.dot(h0, A_ref[...], preferred_element_type=_f32)
    ts_ref[:, : _H // 2] = _pack_cols(
        jnp.dot(h0, B_ref[...], preferred_element_type=_f32))
    ts_ref[:, _H // 2:] = _pack_cols(
        jnp.dot(h0, V_ref[...], preferred_element_type=_f32))
    hu_ref[...] = jnp.dot(h0, U_ref[...], preferred_element_type=_f32)


def _proj_body(h_ref, Wn_ref, A_ref, B_ref, V_ref, U_ref,
               h0_ref, td_ref, ts_ref, hu_ref):
    h0 = jnp.dot(h_ref[...], Wn_ref[...], preferred_element_type=_f32)
    h0_ref[...] = h0
    _proj_tables(h0, A_ref, B_ref, V_ref, U_ref, td_ref, ts_ref, hu_ref)


def _tc_proj(h, Wn, A0, B0, V0, U0):
    n_blk = pl.BlockSpec((_BN, _H), lambda i: (i, 0))
    w_blk = pl.BlockSpec((_H, _H), lambda i: (0, 0))
    return pl.pallas_call(
        _proj_body,
        grid=(_N // _BN,),
        in_specs=[n_blk, w_blk, w_blk, w_blk, w_blk, w_blk],
        out_specs=[n_blk, n_blk, pl.BlockSpec((_BN, _H), lambda i: (i, 0)),
                   n_blk],
        out_shape=[
            jax.ShapeDtypeStruct((_N, _H), _f32),
            jax.ShapeDtypeStruct((_N, _H), _f32),
            jax.ShapeDtypeStruct((_N, _H), jnp.int32),
            jax.ShapeDtypeStruct((_N, _H), _f32),
        ],
    )(h, Wn, A0, B0, V0, U0)


def _upd_proj_body(h_ref, na_ref, nb_ref, da_ref, db_ref, hu_ref,
                   A_ref, B_ref, V_ref, U_ref,
                   h1_ref, td_ref, ts_ref, hu1_ref):
    num = na_ref[...] + nb_ref[...]
    den = da_ref[...] + db_ref[...]
    h1 = h_ref[...] + jnp.maximum(hu_ref[...] + num / (den + 1e-6), 0.0)
    h1_ref[...] = h1
    _proj_tables(h1, A_ref, B_ref, V_ref, U_ref, td_ref, ts_ref, hu1_ref)


def _tc_upd_proj(h0, numa, numb, dena, denb, hu0, A1, B1, V1, U1):
    n_blk = pl.BlockSpec((_BN, _H), lambda i: (i, 0))
    w_blk = pl.BlockSpec((_H, _H), lambda i: (0, 0))
    return pl.pallas_call(
        _upd_proj_body,
        grid=(_N // _BN,),
        in_specs=[n_blk, n_blk, n_blk, n_blk, n_blk, n_blk,
                  w_blk, w_blk, w_blk, w_blk],
        out_specs=[n_blk, n_blk, pl.BlockSpec((_BN, _H), lambda i: (i, 0)),
                   n_blk],
        out_shape=[
            jax.ShapeDtypeStruct((_N, _H), _f32),
            jax.ShapeDtypeStruct((_N, _H), _f32),
            jax.ShapeDtypeStruct((_N, _H), jnp.int32),
            jax.ShapeDtypeStruct((_N, _H), _f32),
        ],
    )(h0, numa, numb, dena, denb, hu0, A1, B1, V1, U1)


def _mid0_body(er_ref, gA_ref, gBV_ref, We_ref, C0_ref, C1_ref,
               msg_ref, sig_ref, eC1_ref):
    P0 = jnp.dot(We_ref[...], C0_ref[...], preferred_element_type=_f32)
    P1 = jnp.dot(We_ref[...], C1_ref[...], preferred_element_type=_f32)
    er = er_ref[...]
    ehat = (jnp.dot(er, P0, preferred_element_type=_f32)
            + gA_ref[...] + _unpack_cols(gBV_ref[:, : _H // 2]))
    sig = jax.nn.sigmoid(ehat)
    sig_ref[...] = sig
    msg_ref[...] = sig * _unpack_cols(gBV_ref[:, _H // 2:])
    r = jnp.maximum(ehat, 0.0)
    eC1_ref[...] = (jnp.dot(er, P1, preferred_element_type=_f32)
                    + jnp.dot(r, C1_ref[...],
                              preferred_element_type=_f32)).astype(_bf16)


def _make_mid0(half):
    off = half * (_EH // _BE)
    e_blk = pl.BlockSpec((_BE, _H), lambda i: (i, 0))

    def call(e_raw, gA, gBV, We, C0, C1):
        return pl.pallas_call(
            _mid0_body,
            grid=(_EH // _BE,),
            in_specs=[
                pl.BlockSpec((_BE, _DE), lambda i: (i + off, 0)),
                e_blk,
                pl.BlockSpec((_BE, _H), lambda i: (i, 0)),
                pl.BlockSpec((_DE, _H), lambda i: (0, 0)),
                pl.BlockSpec((_H, _H), lambda i: (0, 0)),
                pl.BlockSpec((_H, _H), lambda i: (0, 0)),
            ],
            out_specs=[e_blk, e_blk, e_blk],
            out_shape=[
                jax.ShapeDtypeStruct((_EH, _H), _f32),
                jax.ShapeDtypeStruct((_EH, _H), _f32),
                jax.ShapeDtypeStruct((_EH, _H), _bf16),
            ],
        )(e_raw, gA, gBV, We, C0, C1)

    return call


_tc_mid0_h = (_make_mid0(0), _make_mid0(1))


def _mid1_body(eC1_ref, gA_ref, gBV_ref, msg_ref, sig_ref):
    ehat = (eC1_ref[...].astype(_f32) + gA_ref[...]
            + _unpack_cols(gBV_ref[:, : _H // 2]))
    sig = jax.nn.sigmoid(ehat)
    sig_ref[...] = sig
    msg_ref[...] = sig * _unpack_cols(gBV_ref[:, _H // 2:])


def _tc_mid1(eC1_h, gA, gBV):
    e_blk = pl.BlockSpec((_BE, _H), lambda i: (i, 0))
    return pl.pallas_call(
        _mid1_body,
        grid=(_EH // _BE,),
        in_specs=[e_blk, e_blk, pl.BlockSpec((_BE, _H), lambda i: (i, 0))],
        out_specs=[e_blk, e_blk],
        out_shape=[
            jax.ShapeDtypeStruct((_EH, _H), _f32),
            jax.ShapeDtypeStruct((_EH, _H), _f32),
        ],
    )(eC1_h, gA, gBV)


def _head_body(h_ref, na_ref, nb_ref, da_ref, db_ref, hu_ref, act_ref,
               W1h_ref, W1a_ref, b1_ref, W2_ref, b2_ref, out_ref):
    i = pl.program_id(0)
    num = na_ref[...] + nb_ref[...]
    den = da_ref[...] + db_ref[...]
    h2 = h_ref[...] + jnp.maximum(hu_ref[...] + num / (den + 1e-6), 0.0)
    z = jnp.maximum(
        jnp.dot(h2, W1h_ref[...], preferred_element_type=_f32)
        + jnp.dot(act_ref[...], W1a_ref[...], preferred_element_type=_f32)
        + b1_ref[...], 0.0)
    y = jnp.dot(z, W2_ref[...], preferred_element_type=_f32) + b2_ref[...]

    @pl.when(i == 0)
    def _():
        out_ref[...] = jnp.zeros_like(out_ref)

    out_ref[...] += jnp.reshape(jnp.sum(y) / _N, (1, 1))


def _tc_head(h1, numa, numb, dena, denb, hu1, action, W1h, W1a, b1, W2, b2):
    n_blk = pl.BlockSpec((_BN, _H), lambda i: (i, 0))
    return pl.pallas_call(
        _head_body,
        grid=(_N // _BN,),
        in_specs=[
            n_blk, n_blk, n_blk, n_blk, n_blk, n_blk,
            pl.BlockSpec((_BN, _AD), lambda i: (i, 0)),
            pl.BlockSpec((_H, _H), lambda i: (0, 0)),
            pl.BlockSpec((_AD, _H), lambda i: (0, 0)),
            pl.BlockSpec((1, _H), lambda i: (0, 0)),
            pl.BlockSpec((_H, 1), lambda i: (0, 0)),
            pl.BlockSpec((1, 1), lambda i: (0, 0)),
        ],
        out_specs=pl.BlockSpec((1, 1), lambda i: (0, 0)),
        out_shape=jax.ShapeDtypeStruct((1, 1), _f32),
    )(h1, numa, numb, dena, denb, hu1, action, W1h, W1a, b1, W2, b2)


_MESH = plsc.VectorSubcoreMesh(core_axis_name="c", subcore_axis_name="s")

_GK = 5
_GCH = 40
_GSB = _GK * _GCH


def _make_gather(eoff, esz):
    ew = esz // _NW
    nsb = ew // _GSB

    def body(td_hbm, ts_hbm, dst_hbm, src_hbm, gA_hbm, gBV_hbm,
             idx_d0, idx_s0, bufA0, bufBV0,
             idx_d1, idx_s1, bufA1, bufBV1, semi, semg, semw):
        c = lax.axis_index("c")
        s = lax.axis_index("s")
        wid = s * _NC + c
        w0in = eoff + wid * ew
        w0out = wid * ew

        def start_idx(i, idx_d, idx_s):
            pltpu.async_copy(dst_hbm.at[pl.ds(w0in + i * _GSB, _GSB)], idx_d,
                             semi)
            pltpu.async_copy(src_hbm.at[pl.ds(w0in + i * _GSB, _GSB)], idx_s,
                             semi)

        def wait_idx(idx_d, idx_s):
            pltpu.make_async_copy(dst_hbm.at[pl.ds(0, _GSB)], idx_d,
                                  semi).wait()
            pltpu.make_async_copy(src_hbm.at[pl.ds(0, _GSB)], idx_s,
                                  semi).wait()

        def start_gathers(idx_d, idx_s, bufA, bufBV):
            for k in range(_GK):
                sl = pl.ds(k * _GCH, _GCH)
                pltpu.async_copy(td_hbm.at[idx_d.at[sl]], bufA.at[sl], semg)
                pltpu.async_copy(ts_hbm.at[idx_s.at[sl]], bufBV.at[sl], semg)

        def wait_gathers(idx_d, idx_s, bufA, bufBV):
            for k in range(_GK):
                sl = pl.ds(k * _GCH, _GCH)
                pltpu.make_async_copy(td_hbm.at[idx_d.at[sl]], bufA.at[sl],
                                      semg).wait()
                pltpu.make_async_copy(ts_hbm.at[idx_s.at[sl]], bufBV.at[sl],
                                      semg).wait()

        def start_writes(i, bufA, bufBV):
            pltpu.async_copy(bufA, gA_hbm.at[pl.ds(w0out + i * _GSB, _GSB)],
                             semw)
            pltpu.async_copy(bufBV, gBV_hbm.at[pl.ds(w0out + i * _GSB, _GSB)],
                             semw)

        def wait_writes(bufA, bufBV):
            pltpu.make_async_copy(bufA, gA_hbm.at[pl.ds(0, _GSB)],
                                  semw).wait()
            pltpu.make_async_copy(bufBV, gBV_hbm.at[pl.ds(0, _GSB)],
                                  semw).wait()

        start_idx(0, idx_d0, idx_s0)

        def pair(j, carry):
            i0 = 2 * j
            wait_idx(idx_d0, idx_s0)
            start_gathers(idx_d0, idx_s0, bufA0, bufBV0)
            start_idx(i0 + 1, idx_d1, idx_s1)
            wait_gathers(idx_d0, idx_s0, bufA0, bufBV0)
            start_writes(i0, bufA0, bufBV0)
            wait_idx(idx_d1, idx_s1)
            start_gathers(idx_d1, idx_s1, bufA1, bufBV1)

            @pl.when(i0 + 2 < nsb)
            def _():
                start_idx(i0 + 2, idx_d0, idx_s0)

            wait_gathers(idx_d1, idx_s1, bufA1, bufBV1)
            wait_writes(bufA0, bufBV0)
            start_writes(i0 + 1, bufA1, bufBV1)
            wait_writes(bufA1, bufBV1)
            return carry

        lax.fori_loop(0, nsb // 2, pair, 0)
        if nsb % 2:
            wait_idx(idx_d0, idx_s0)
            start_gathers(idx_d0, idx_s0, bufA0, bufBV0)
            wait_gathers(idx_d0, idx_s0, bufA0, bufBV0)
            start_writes(nsb - 1, bufA0, bufBV0)
            wait_writes(bufA0, bufBV0)

    return pl.kernel(
        body,
        out_type=[
            jax.ShapeDtypeStruct((esz, _H), _f32),
            jax.ShapeDtypeStruct((esz, _H), jnp.int32),
        ],
        mesh=_MESH,
        scratch_types=[
            pltpu.VMEM((_GSB,), jnp.int32),
            pltpu.VMEM((_GSB,), jnp.int32),
            pltpu.VMEM((_GSB, _H), _f32),
            pltpu.VMEM((_GSB, _H), jnp.int32),
            pltpu.VMEM((_GSB,), jnp.int32),
            pltpu.VMEM((_GSB,), jnp.int32),
            pltpu.VMEM((_GSB, _H), _f32),
            pltpu.VMEM((_GSB, _H), jnp.int32),
            pltpu.SemaphoreType.DMA,
            pltpu.SemaphoreType.DMA,
            pltpu.SemaphoreType.DMA,
        ],
    )


_sc_gather_h = (_make_gather(0, _EH), _make_gather(_EH, _EH))

_SK = 1
_SCH = 80
_SSB = _SK * _SCH


def _make_scatter(eoff, esz):
    ew = esz // _NS
    nsb = ew // _SSB
    rows = _NP // _NS

    def stream(data_hbm, dst3_hbm, acc, dbuf0, idx0, dbuf1, idx1,
               semd, sems, s):
        base0 = s * ew

        def start_load(i, dbuf, idxbuf):
            pltpu.async_copy(data_hbm.at[pl.ds(base0 + i * _SSB, _SSB)], dbuf,
                             semd)
            pltpu.async_copy(
                dst3_hbm.at[pl.ds((eoff + base0 + i * _SSB) // _SCH, _SK)],
                idxbuf, semd)

        def wait_load(dbuf, idxbuf):
            pltpu.make_async_copy(data_hbm.at[pl.ds(0, _SSB)], dbuf,
                                  semd).wait()
            pltpu.make_async_copy(dst3_hbm.at[pl.ds(0, _SK)], idxbuf,
                                  semd).wait()

        def scat(dbuf, idxbuf):
            for k in range(_SK):
                pltpu.async_copy(dbuf.at[pl.ds(k * _SCH, _SCH)],
                                 acc.at[idxbuf.at[k, 0]], sems, add=True)
            for k in range(_SK):
                pltpu.make_async_copy(dbuf.at[pl.ds(k * _SCH, _SCH)],
                                      acc.at[idxbuf.at[k, 0]], sems).wait()

        start_load(0, dbuf0, idx0)

        def pairfn(j, carry):
            i0 = 2 * j
            wait_load(dbuf0, idx0)
            start_load(i0 + 1, dbuf1, idx1)
            scat(dbuf0, idx0)

            @pl.when(i0 + 2 < nsb)
            def _():
                start_load(i0 + 2, dbuf0, idx0)

            wait_load(dbuf1, idx1)
            scat(dbuf1, idx1)
            return carry

        lax.fori_loop(0, nsb // 2, pairfn, 0)
        if nsb % 2:
            wait_load(dbuf0, idx0)
            scat(dbuf0, idx0)

    def body(msg_hbm, sig_hbm, dst3_hbm, zeros_hbm, num_hbm, den_hbm,
             dbuf0, idx0, dbuf1, idx1, acc, semd, sems):
        c = lax.axis_index("c")
        s = lax.axis_index("s")
        pltpu.sync_copy(zeros_hbm.at[pl.ds(s * rows, rows)],
                        acc.at[pl.ds(s * rows, rows)])
        plsc.subcore_barrier()

        @pl.when(c == 0)
        def _():
            stream(msg_hbm, dst3_hbm, acc, dbuf0, idx0, dbuf1, idx1,
                   semd, sems, s)

        @pl.when(c == 1)
        def _():
            stream(sig_hbm, dst3_hbm, acc, dbuf0, idx0, dbuf1, idx1,
                   semd, sems, s)

        plsc.subcore_barrier()

        @pl.when(c == 0)
        def _():
            pltpu.sync_copy(acc.at[pl.ds(s * rows, rows)],
                            num_hbm.at[pl.ds(s * rows, rows)])

        @pl.when(c == 1)
        def _():
            pltpu.sync_copy(acc.at[pl.ds(s * rows, rows)],
                            den_hbm.at[pl.ds(s * rows, rows)])

    return pl.kernel(
        body,
        out_type=[
            jax.ShapeDtypeStruct((_NP, _H), _f32),
            jax.ShapeDtypeStruct((_NP, _H), _f32),
        ],
        mesh=_MESH,
        scratch_types=[
            pltpu.VMEM((_SSB, _H), _f32),
            pltpu.VMEM((_SK, 1, _SCH), jnp.int32),
            pltpu.VMEM((_SSB, _H), _f32),
            pltpu.VMEM((_SK, 1, _SCH), jnp.int32),
            pltpu.VMEM_SHARED((_NP, _H), _f32),
            pltpu.SemaphoreType.DMA,
            pltpu.SemaphoreType.DMA,
        ],
    )


_sc_scatter_h = (_make_scatter(0, _EH), _make_scatter(_EH, _EH))


def _layer_edges(e_raw, eC1_h, td, ts, dst, src, dst3, zeros_n, We, C0, C1,
                 layer):
    gA = [None, None]
    gBV = [None, None]
    out = [None, None]
    nd = [None, None]
    gA[0], gBV[0] = _sc_gather_h[0](td, ts, dst, src)
    for hf in (0, 1):
        if hf == 0:
            gA[1], gBV[1] = _sc_gather_h[1](td, ts, dst, src)
        if layer == 0:
            out[hf] = _tc_mid0_h[hf](e_raw, gA[hf], gBV[hf], We, C0, C1)
        else:
            out[hf] = _tc_mid1(eC1_h[hf], gA[hf], gBV[hf])
        nd[hf] = _sc_scatter_h[hf](out[hf][0], out[hf][1], dst3, zeros_n)
    if layer == 0:
        return (nd[0][0], nd[1][0], nd[0][1], nd[1][1],
                (out[0][2], out[1][2]))
    return nd[0][0], nd[1][0], nd[0][1], nd[1][1]


def kernel(h, e, edge_index, action, Wn, We, A, B, C, U, V, W1, b1, W2, b2):
    src = edge_index[0]
    dst = edge_index[1]
    dst3 = dst.reshape(_E // _SCH, 1, _SCH)
    zeros_n = jnp.zeros((_NP, _H), _f32)

    h0, td0, ts0, hu0 = _tc_proj(h, Wn, A[0], B[0], V[0], U[0])
    na0, nb0, da0, db0, eC1_h = _layer_edges(
        e, None, td0, ts0, dst, src, dst3, zeros_n, We, C[0], C[1], 0)

    h1, td1, ts1, hu1 = _tc_upd_proj(h0, na0, nb0, da0, db0, hu0,
                                     A[1], B[1], V[1], U[1])
    na1, nb1, da1, db1 = _layer_edges(
        None, eC1_h, td1, ts1, dst, src, dst3, zeros_n, We, C[0], C[1], 1)

    return _tc_head(h1, na1, nb1, da1, db1, hu1, action,
                    W1[:_H], W1[_H:], b1.reshape(1, _H),
                    W2, b2.reshape(1, 1))


# speedup vs baseline: 3.0170x; 1.0047x over previous
import functools

import jax
import jax.numpy as jnp
import numpy as np
from jax import lax
from jax.experimental import pallas as pl
from jax.experimental.pallas import tpu as pltpu
from jax.experimental.pallas import tpu_sc as plsc

_N = 10000
_NP = 10240
_E = 320000
_EH = _E // 2
_H = 128
_DE = 16
_AD = 8

_SCI = plsc.get_sparse_core_info()
_NC = _SCI.num_cores
_NS = _SCI.num_subcores
_NW = _NC * _NS

_BN = 2000
_BE = 2000

_f32 = jnp.float32
_bf16 = jnp.bfloat16


_HI_MASK = np.uint32(0xFFFF0000)


def _pack_cols(x):
    u = jax.lax.bitcast_convert_type(x, jnp.uint32)
    r = (u + np.uint32(0x7FFF) + ((u >> 16) & np.uint32(1))) & _HI_MASK
    lo = r[:, : _H // 2]
    hi = r[:, _H // 2:]
    return jax.lax.bitcast_convert_type(hi | (lo >> 16), jnp.int32)


def _unpack_cols(xi):
    u = jax.lax.bitcast_convert_type(xi, jnp.uint32)
    lo = jax.lax.bitcast_convert_type(u << 16, _f32)
    hi = jax.lax.bitcast_convert_type(u & _HI_MASK, _f32)
    return jnp.concatenate([lo, hi], axis=1)


def _proj_tables(h0, A_ref, B_ref, V_ref, U_ref, td_ref, ts_ref, hu_ref):
    td_ref[...] = jnp.dot(h0, A_ref[...], preferred_element_type=_f32)
    ts_ref[:, : _H // 2] = _pack_cols(
        jnp.dot(h0, B_ref[...], preferred_element_type=_f32))
    ts_ref[:, _H // 2:] = _pack_cols(
        jnp.dot(h0, V_ref[...], preferred_element_type=_f32))
    hu_ref[...] = jnp.dot(h0, U_ref[...], preferred_element_type=_f32)


def _proj_body(h_ref, Wn_ref, A_ref, B_ref, V_ref, U_ref,
               h0_ref, td_ref, ts_ref, hu_ref):
    h0 = jnp.dot(h_ref[...], Wn_ref[...], preferred_element_type=_f32)
    h0_ref[...] = h0
    _proj_tables(h0, A_ref, B_ref, V_ref, U_ref, td_ref, ts_ref, hu_ref)


def _tc_proj(h, Wn, A0, B0, V0, U0):
    n_blk = pl.BlockSpec((_BN, _H), lambda i: (i, 0))
    w_blk = pl.BlockSpec((_H, _H), lambda i: (0, 0))
    return pl.pallas_call(
        _proj_body,
        grid=(_N // _BN,),
        in_specs=[n_blk, w_blk, w_blk, w_blk, w_blk, w_blk],
        out_specs=[n_blk, n_blk, pl.BlockSpec((_BN, _H), lambda i: (i, 0)),
                   n_blk],
        out_shape=[
            jax.ShapeDtypeStruct((_N, _H), _f32),
            jax.ShapeDtypeStruct((_N, _H), _f32),
            jax.ShapeDtypeStruct((_N, _H), jnp.int32),
            jax.ShapeDtypeStruct((_N, _H), _f32),
        ],
    )(h, Wn, A0, B0, V0, U0)


def _upd_proj_body(h_ref, na_ref, nb_ref, da_ref, db_ref, hu_ref,
                   A_ref, B_ref, V_ref, U_ref,
                   h1_ref, td_ref, ts_ref, hu1_ref):
    num = na_ref[...] + nb_ref[...]
    den = da_ref[...] + db_ref[...]
    h1 = h_ref[...] + jnp.maximum(hu_ref[...] + num / (den + 1e-6), 0.0)
    h1_ref[...] = h1
    _proj_tables(h1, A_ref, B_ref, V_ref, U_ref, td_ref, ts_ref, hu1_ref)


def _tc_upd_proj(h0, numa, numb, dena, denb, hu0, A1, B1, V1, U1):
    n_blk = pl.BlockSpec((_BN, _H), lambda i: (i, 0))
    w_blk = pl.BlockSpec((_H, _H), lambda i: (0, 0))
    return pl.pallas_call(
        _upd_proj_body,
        grid=(_N // _BN,),
        in_specs=[n_blk, n_blk, n_blk, n_blk, n_blk, n_blk,
                  w_blk, w_blk, w_blk, w_blk],
        out_specs=[n_blk, n_blk, pl.BlockSpec((_BN, _H), lambda i: (i, 0)),
                   n_blk],
        out_shape=[
            jax.ShapeDtypeStruct((_N, _H), _f32),
            jax.ShapeDtypeStruct((_N, _H), _f32),
            jax.ShapeDtypeStruct((_N, _H), jnp.int32),
            jax.ShapeDtypeStruct((_N, _H), _f32),
        ],
    )(h0, numa, numb, dena, denb, hu0, A1, B1, V1, U1)


def _mid0_body(er_ref, gA_ref, gBV_ref, We_ref, C0_ref, C1_ref,
               msg_ref, sig_ref, eC1_ref):
    P0 = jnp.dot(We_ref[...], C0_ref[...], preferred_element_type=_f32)
    P1 = jnp.dot(We_ref[...], C1_ref[...], preferred_element_type=_f32)
    er = er_ref[...]
    ehat = (jnp.dot(er, P0, preferred_element_type=_f32)
            + gA_ref[...] + _unpack_cols(gBV_ref[:, : _H // 2]))
    sig = jax.nn.sigmoid(ehat)
    sig_ref[...] = sig
    msg_ref[...] = sig * _unpack_cols(gBV_ref[:, _H // 2:])
    r = jnp.maximum(ehat, 0.0)
    eC1_ref[...] = (jnp.dot(er, P1, preferred_element_type=_f32)
                    + jnp.dot(r, C1_ref[...],
                              preferred_element_type=_f32)).astype(_bf16)


def _make_mid0(half):
    off = half * (_EH // _BE)
    e_blk = pl.BlockSpec((_BE, _H), lambda i: (i, 0))

    def call(e_raw, gA, gBV, We, C0, C1):
        return pl.pallas_call(
            _mid0_body,
            grid=(_EH // _BE,),
            in_specs=[
                pl.BlockSpec((_BE, _DE), lambda i: (i + off, 0)),
                e_blk,
                pl.BlockSpec((_BE, _H), lambda i: (i, 0)),
                pl.BlockSpec((_DE, _H), lambda i: (0, 0)),
                pl.BlockSpec((_H, _H), lambda i: (0, 0)),
                pl.BlockSpec((_H, _H), lambda i: (0, 0)),
            ],
            out_specs=[e_blk, e_blk, e_blk],
            out_shape=[
                jax.ShapeDtypeStruct((_EH, _H), _f32),
                jax.ShapeDtypeStruct((_EH, _H), _f32),
                jax.ShapeDtypeStruct((_EH, _H), _bf16),
            ],
        )(e_raw, gA, gBV, We, C0, C1)

    return call


_tc_mid0_h = (_make_mid0(0), _make_mid0(1))


def _mid1_body(eC1_ref, gA_ref, gBV_ref, msg_ref, sig_ref):
    ehat = (eC1_ref[...].astype(_f32) + gA_ref[...]
            + _unpack_cols(gBV_ref[:, : _H // 2]))
    sig = jax.nn.sigmoid(ehat)
    sig_ref[...] = sig
    msg_ref[...] = sig * _unpack_cols(gBV_ref[:, _H // 2:])


def _tc_mid1(eC1_h, gA, gBV):
    e_blk = pl.BlockSpec((_BE, _H), lambda i: (i, 0))
    return pl.pallas_call(
        _mid1_body,
        grid=(_EH // _BE,),
        in_specs=[e_blk, e_blk, pl.BlockSpec((_BE, _H), lambda i: (i, 0))],
        out_specs=[e_blk, e_blk],
        out_shape=[
            jax.ShapeDtypeStruct((_EH, _H), _f32),
            jax.ShapeDtypeStruct((_EH, _H), _f32),
        ],
    )(eC1_h, gA, gBV)


def _head_body(h_ref, na_ref, nb_ref, da_ref, db_ref, hu_ref, act_ref,
               W1h_ref, W1a_ref, b1_ref, W2_ref, b2_ref, out_ref):
    i = pl.program_id(0)
    num = na_ref[...] + nb_ref[...]
    den = da_ref[...] + db_ref[...]
    h2 = h_ref[...] + jnp.maximum(hu_ref[...] + num / (den + 1e-6), 0.0)
    z = jnp.maximum(
        jnp.dot(h2, W1h_ref[...], preferred_element_type=_f32)
        + jnp.dot(act_ref[...], W1a_ref[...], preferred_element_type=_f32)
        + b1_ref[...], 0.0)
    y = jnp.dot(z, W2_ref[...], preferred_element_type=_f32) + b2_ref[...]

    @pl.when(i == 0)
    def _():
        out_ref[...] = jnp.zeros_like(out_ref)

    out_ref[...] += jnp.reshape(jnp.sum(y) / _N, (1, 1))


def _tc_head(h1, numa, numb, dena, denb, hu1, action, W1h, W1a, b1, W2, b2):
    n_blk = pl.BlockSpec((_BN, _H), lambda i: (i, 0))
    return pl.pallas_call(
        _head_body,
        grid=(_N // _BN,),
        in_specs=[
            n_blk, n_blk, n_blk, n_blk, n_blk, n_blk,
            pl.BlockSpec((_BN, _AD), lambda i: (i, 0)),
            pl.BlockSpec((_H, _H), lambda i: (0, 0)),
            pl.BlockSpec((_AD, _H), lambda i: (0, 0)),
            pl.BlockSpec((1, _H), lambda i: (0, 0)),
            pl.BlockSpec((_H, 1), lambda i: (0, 0)),
            pl.BlockSpec((1, 1), lambda i: (0, 0)),
        ],
        out_specs=pl.BlockSpec((1, 1), lambda i: (0, 0)),
        out_shape=jax.ShapeDtypeStruct((1, 1), _f32),
    )(h1, numa, numb, dena, denb, hu1, action, W1h, W1a, b1, W2, b2)


_MESH = plsc.VectorSubcoreMesh(core_axis_name="c", subcore_axis_name="s")

_GK = 5
_GCH = 40
_GSB = _GK * _GCH


def _make_gather(eoff, esz):
    ew = esz // _NW
    nsb = ew // _GSB

    def body(td_hbm, ts_hbm, dst_hbm, src_hbm, gA_hbm, gBV_hbm,
             idx_d0, idx_s0, bufA0, bufBV0,
             idx_d1, idx_s1, bufA1, bufBV1, semi, semg, semw):
        c = lax.axis_index("c")
        s = lax.axis_index("s")
        wid = s * _NC + c
        w0in = eoff + wid * ew
        w0out = wid * ew

        def start_idx(i, idx_d, idx_s):
            pltpu.async_copy(dst_hbm.at[pl.ds(w0in + i * _GSB, _GSB)], idx_d,
                             semi)
            pltpu.async_copy(src_hbm.at[pl.ds(w0in + i * _GSB, _GSB)], idx_s,
                             semi)

        def wait_idx(idx_d, idx_s):
            pltpu.make_async_copy(dst_hbm.at[pl.ds(0, _GSB)], idx_d,
                                  semi).wait()
            pltpu.make_async_copy(src_hbm.at[pl.ds(0, _GSB)], idx_s,
                                  semi).wait()

        def start_gathers(idx_d, idx_s, bufA, bufBV):
            for k in range(_GK):
                sl = pl.ds(k * _GCH, _GCH)
                pltpu.async_copy(td_hbm.at[idx_d.at[sl]], bufA.at[sl], semg)
                pltpu.async_copy(ts_hbm.at[idx_s.at[sl]], bufBV.at[sl], semg)

        def wait_gathers(idx_d, idx_s, bufA, bufBV):
            for k in range(_GK):
                sl = pl.ds(k * _GCH, _GCH)
                pltpu.make_async_copy(td_hbm.at[idx_d.at[sl]], bufA.at[sl],
                                      semg).wait()
                pltpu.make_async_copy(ts_hbm.at[idx_s.at[sl]], bufBV.at[sl],
                                      semg).wait()

        def start_writes(i, bufA, bufBV):
            pltpu.async_copy(bufA, gA_hbm.at[pl.ds(w0out + i * _GSB, _GSB)],
                             semw)
            pltpu.async_copy(bufBV, gBV_hbm.at[pl.ds(w0out + i * _GSB, _GSB)],
                             semw)

        def wait_writes(bufA, bufBV):
            pltpu.make_async_copy(bufA, gA_hbm.at[pl.ds(0, _GSB)],
                                  semw).wait()
            pltpu.make_async_copy(bufBV, gBV_hbm.at[pl.ds(0, _GSB)],
                                  semw).wait()

        start_idx(0, idx_d0, idx_s0)

        def pair(j, carry):
            i0 = 2 * j
            wait_idx(idx_d0, idx_s0)

            @pl.when(j > 0)
            def _():
                wait_writes(bufA0, bufBV0)

            start_gathers(idx_d0, idx_s0, bufA0, bufBV0)
            start_idx(i0 + 1, idx_d1, idx_s1)
            wait_gathers(idx_d0, idx_s0, bufA0, bufBV0)
            start_writes(i0, bufA0, bufBV0)
            wait_idx(idx_d1, idx_s1)

            @pl.when(j > 0)
            def _():
                wait_writes(bufA1, bufBV1)

            start_gathers(idx_d1, idx_s1, bufA1, bufBV1)

            @pl.when(i0 + 2 < nsb)
            def _():
                start_idx(i0 + 2, idx_d0, idx_s0)

            wait_gathers(idx_d1, idx_s1, bufA1, bufBV1)
            start_writes(i0 + 1, bufA1, bufBV1)
            return carry

        lax.fori_loop(0, nsb // 2, pair, 0)
        wait_writes(bufA0, bufBV0)
        wait_writes(bufA1, bufBV1)
        if nsb % 2:
            wait_idx(idx_d0, idx_s0)
            start_gathers(idx_d0, idx_s0, bufA0, bufBV0)
            wait_gathers(idx_d0, idx_s0, bufA0, bufBV0)
            start_writes(nsb - 1, bufA0, bufBV0)
            wait_writes(bufA0, bufBV0)

    return pl.kernel(
        body,
        out_type=[
            jax.ShapeDtypeStruct((esz, _H), _f32),
            jax.ShapeDtypeStruct((esz, _H), jnp.int32),
        ],
        mesh=_MESH,
        scratch_types=[
            pltpu.VMEM((_GSB,), jnp.int32),
            pltpu.VMEM((_GSB,), jnp.int32),
            pltpu.VMEM((_GSB, _H), _f32),
            pltpu.VMEM((_GSB, _H), jnp.int32),
            pltpu.VMEM((_GSB,), jnp.int32),
            pltpu.VMEM((_GSB,), jnp.int32),
            pltpu.VMEM((_GSB, _H), _f32),
            pltpu.VMEM((_GSB, _H), jnp.int32),
            pltpu.SemaphoreType.DMA,
            pltpu.SemaphoreType.DMA,
            pltpu.SemaphoreType.DMA,
        ],
    )


_sc_gather_h = (_make_gather(0, _EH), _make_gather(_EH, _EH))

_SK = 1
_SCH = 80
_SSB = _SK * _SCH


def _make_scatter(eoff, esz):
    ew = esz // _NS
    nsb = ew // _SSB
    rows = _NP // _NS

    def stream(data_hbm, dst3_hbm, acc, dbuf0, idx0, dbuf1, idx1,
               semd, sems, s):
        base0 = s * ew

        def start_load(i, dbuf, idxbuf):
            pltpu.async_copy(data_hbm.at[pl.ds(base0 + i * _SSB, _SSB)], dbuf,
                             semd)
            pltpu.async_copy(
                dst3_hbm.at[pl.ds((eoff + base0 + i * _SSB) // _SCH, _SK)],
                idxbuf, semd)

        def wait_load(dbuf, idxbuf):
            pltpu.make_async_copy(data_hbm.at[pl.ds(0, _SSB)], dbuf,
                                  semd).wait()
            pltpu.make_async_copy(dst3_hbm.at[pl.ds(0, _SK)], idxbuf,
                                  semd).wait()

        def scat(dbuf, idxbuf):
            for k in range(_SK):
                pltpu.async_copy(dbuf.at[pl.ds(k * _SCH, _SCH)],
                                 acc.at[idxbuf.at[k, 0]], sems, add=True)
            for k in range(_SK):
                pltpu.make_async_copy(dbuf.at[pl.ds(k * _SCH, _SCH)],
                                      acc.at[idxbuf.at[k, 0]], sems).wait()

        def pairfn(j, carry):
            i0 = 2 * j
            wait_load(dbuf0, idx0)
            start_load(i0 + 1, dbuf1, idx1)
            scat(dbuf0, idx0)

            @pl.when(i0 + 2 < nsb)
            def _():
                start_load(i0 + 2, dbuf0, idx0)

            wait_load(dbuf1, idx1)
            scat(dbuf1, idx1)
            return carry

        lax.fori_loop(0, nsb // 2, pairfn, 0)
        if nsb % 2:
            wait_load(dbuf0, idx0)
            scat(dbuf0, idx0)

    def start_first_load(data_hbm, dst3_hbm, dbuf, idxbuf, semd, s):
        base0 = s * ew
        pltpu.async_copy(data_hbm.at[pl.ds(base0, _SSB)], dbuf, semd)
        pltpu.async_copy(dst3_hbm.at[pl.ds((eoff + base0) // _SCH, _SK)],
                         idxbuf, semd)

    def body(msg_hbm, sig_hbm, dst3_hbm, zeros_hbm, num_hbm, den_hbm,
             dbuf0, idx0, dbuf1, idx1, acc, semd, sems):
        c = lax.axis_index("c")
        s = lax.axis_index("s")

        @pl.when(c == 0)
        def _():
            start_first_load(msg_hbm, dst3_hbm, dbuf0, idx0, semd, s)

        @pl.when(c == 1)
        def _():
            start_first_load(sig_hbm, dst3_hbm, dbuf0, idx0, semd, s)

        pltpu.sync_copy(zeros_hbm.at[pl.ds(s * rows, rows)],
                        acc.at[pl.ds(s * rows, rows)])
        plsc.subcore_barrier()

        @pl.when(c == 0)
        def _():
            stream(msg_hbm, dst3_hbm, acc, dbuf0, idx0, dbuf1, idx1,
                   semd, sems, s)

        @pl.when(c == 1)
        def _():
            stream(sig_hbm, dst3_hbm, acc, dbuf0, idx0, dbuf1, idx1,
                   semd, sems, s)

        plsc.subcore_barrier()

        @pl.when(c == 0)
        def _():
            pltpu.sync_copy(acc.at[pl.ds(s * rows, rows)],
                            num_hbm.at[pl.ds(s * rows, rows)])

        @pl.when(c == 1)
        def _():
            pltpu.sync_copy(acc.at[pl.ds(s * rows, rows)],
                            den_hbm.at[pl.ds(s * rows, rows)])

    return pl.kernel(
        body,
        out_type=[
            jax.ShapeDtypeStruct((_NP, _H), _f32),
            jax.ShapeDtypeStruct((_NP, _H), _f32),
        ],
        mesh=_MESH,
        scratch_types=[
            pltpu.VMEM((_SSB, _H), _f32),
            pltpu.VMEM((_SK, 1, _SCH), jnp.int32),
            pltpu.VMEM((_SSB, _H), _f32),
            pltpu.VMEM((_SK, 1, _SCH), jnp.int32),
            pltpu.VMEM_SHARED((_NP, _H), _f32),
            pltpu.SemaphoreType.DMA,
            pltpu.SemaphoreType.DMA,
        ],
    )


_sc_scatter_h = (_make_scatter(0, _EH), _make_scatter(_EH, _EH))


def _layer_edges(e_raw, eC1_h, td, ts, dst, src, dst3, zeros_n, We, C0, C1,
                 layer):
    gA = [None, None]
    gBV = [None, None]
    out = [None, None]
    nd = [None, None]
    gA[0], gBV[0] = _sc_gather_h[0](td, ts, dst, src)
    for hf in (0, 1):
        if hf == 0:
            gA[1], gBV[1] = _sc_gather_h[1](td, ts, dst, src)
        if layer == 0:
            out[hf] = _tc_mid0_h[hf](e_raw, gA[hf], gBV[hf], We, C0, C1)
        else:
            out[hf] = _tc_mid1(eC1_h[hf], gA[hf], gBV[hf])
        nd[hf] = _sc_scatter_h[hf](out[hf][0], out[hf][1], dst3, zeros_n)
    if layer == 0:
        return (nd[0][0], nd[1][0], nd[0][1], nd[1][1],
                (out[0][2], out[1][2]))
    return nd[0][0], nd[1][0], nd[0][1], nd[1][1]


def kernel(h, e, edge_index, action, Wn, We, A, B, C, U, V, W1, b1, W2, b2):
    src = edge_index[0]
    dst = edge_index[1]
    dst3 = dst.reshape(_E // _SCH, 1, _SCH)
    zeros_n = jnp.zeros((_NP, _H), _f32)

    h0, td0, ts0, hu0 = _tc_proj(h, Wn, A[0], B[0], V[0], U[0])
    na0, nb0, da0, db0, eC1_h = _layer_edges(
        e, None, td0, ts0, dst, src, dst3, zeros_n, We, C[0], C[1], 0)

    h1, td1, ts1, hu1 = _tc_upd_proj(h0, na0, nb0, da0, db0, hu0,
                                     A[1], B[1], V[1], U[1])
    na1, nb1, da1, db1 = _layer_edges(
        None, eC1_h, td1, ts1, dst, src, dst3, zeros_n, We, C[0], C[1], 1)

    return _tc_head(h1, na1, nb1, da1, db1, hu1, action,
                    W1[:_H], W1[_H:], b1.reshape(1, _H),
                    W2, b2.reshape(1, 1))
